# Initial kernel scaffold; baseline (speedup 1.0000x reference)
import jax
import jax.numpy as jnp
from jax.experimental import pallas as pl


def kernel(edge_feat, node_feat, graph_attr, edge_index, params):
    raise NotImplementedError("write your pallas kernel here")


# speedup vs baseline: 2.1049x; 2.1049x over previous
import functools

import jax
import jax.numpy as jnp
from jax import lax
from jax.experimental import pallas as pl
from jax.experimental.pallas import tpu as pltpu
from jax.experimental.pallas import tpu_sc as plsc

F32 = jnp.float32


def _sp(x):
    return jnp.maximum(x, 0.0) + jnp.log1p(jnp.exp(-jnp.abs(x)))


def _sig(x):
    return 1.0 / (1.0 + jnp.exp(-x))


def _dot(a, b):
    return jnp.dot(a, b, preferred_element_type=F32)


def _enc_body(x_ref, w1_ref, b1_ref, w2_ref, b2_ref, o_ref):
    h = _sp(_dot(x_ref[...], w1_ref[...]) + b1_ref[...])
    o_ref[...] = _sp(_dot(h, w2_ref[...]) + b2_ref[...])


def _encode(x, w1, b1, w2, b2, rows):
    n, d = x.shape
    grid = n // rows
    return pl.pallas_call(
        _enc_body,
        grid=(grid,),
        in_specs=[
            pl.BlockSpec((rows, d), lambda i: (i, 0)),
            pl.BlockSpec(w1.shape, lambda i: (0, 0)),
            pl.BlockSpec(b1.shape, lambda i: (0, 0)),
            pl.BlockSpec(w2.shape, lambda i: (0, 0)),
            pl.BlockSpec(b2.shape, lambda i: (0, 0)),
        ],
        out_specs=pl.BlockSpec((rows, w2.shape[1]), lambda i: (i, 0)),
        out_shape=jax.ShapeDtypeStruct((n, w2.shape[1]), F32),
        interpret=False,
    )(x, w1, b1, w2, b2)


_BATCH = 80
_OCH = 2000
_NW = 32


def _sc_gather(v, src2, dst2, E):
    per_w = E // _NW
    nout = per_w // _OCH
    nb = _OCH // _BATCH
    mesh = plsc.VectorSubcoreMesh(core_axis_name="c", subcore_axis_name="s")

    @functools.partial(
        pl.kernel,
        out_type=[
            jax.ShapeDtypeStruct((E, 32), F32),
            jax.ShapeDtypeStruct((E, 32), F32),
        ],
        mesh=mesh,
        scratch_types=[
            pltpu.VMEM((nb, _BATCH), jnp.int32),
            pltpu.VMEM((_OCH, 32), F32),
            pltpu.SemaphoreType.DMA,
        ],
        compiler_params=pltpu.CompilerParams(use_tc_tiling_on_sc=False),
        interpret=False,
    )
    def k(v_hbm, si_hbm, di_hbm, gi_hbm, gj_hbm, idx_v, rows_v, sem):
        cid = lax.axis_index("c")
        sid = lax.axis_index("s")
        wid = sid * 2 + cid
        for o in range(nout):
            ch = wid * nout + o
            e0 = wid * per_w + o * _OCH

            def gth(j, _):
                pltpu.async_copy(
                    v_hbm.at[idx_v.at[j]],
                    rows_v.at[pl.ds(j * _BATCH, _BATCH)],
                    sem,
                ).wait()
                return 0

            pltpu.sync_copy(si_hbm.at[ch], idx_v)
            lax.fori_loop(0, nb, gth, 0)
            pltpu.sync_copy(rows_v, gi_hbm.at[pl.ds(e0, _OCH)])

            pltpu.sync_copy(di_hbm.at[ch], idx_v)
            lax.fori_loop(0, nb, gth, 0)
            pltpu.sync_copy(rows_v, gj_hbm.at[pl.ds(e0, _OCH)])

    return k(v, src2, dst2)


def _sc_scatter(ec, dst2, zeros32, E, N):
    per_w = E // _NW
    nout = per_w // _OCH
    nb = _OCH // _BATCH
    out_n = N // 10
    mesh = plsc.VectorSubcoreMesh(core_axis_name="c", subcore_axis_name="s")

    @functools.partial(
        pl.kernel,
        out_type=[jax.ShapeDtypeStruct((2, N, 32), F32)],
        mesh=mesh,
        scratch_types=[
            pltpu.VMEM((nb, _BATCH), jnp.int32),
            pltpu.VMEM((_OCH, 32), F32),
            pltpu.VMEM((out_n, 32), F32),
            pltpu.VMEM_SHARED((N, 32), F32),
        ],
        compiler_params=pltpu.CompilerParams(use_tc_tiling_on_sc=False),
        interpret=False,
    )
    def k(ec_hbm, di_hbm, z32_hbm, out_hbm, idx_v, rows_v, zb_v, acc_sh):
        cid = lax.axis_index("c")
        sid = lax.axis_index("s")
        wid = sid * 2 + cid

        @pl.when(sid < 10)
        def _():
            pltpu.sync_copy(z32_hbm, zb_v)
            pltpu.sync_copy(zb_v, acc_sh.at[pl.ds(sid * out_n, out_n)])
        plsc.subcore_barrier()

        for o in range(nout):
            ch = wid * nout + o
            e0 = wid * per_w + o * _OCH
            pltpu.sync_copy(di_hbm.at[ch], idx_v)
            pltpu.sync_copy(ec_hbm.at[pl.ds(e0, _OCH)], rows_v)

            def sct(j, _):
                pltpu.sync_copy(
                    rows_v.at[pl.ds(j * _BATCH, _BATCH)],
                    acc_sh.at[idx_v.at[j]],
                    add=True,
                )
                return 0

            lax.fori_loop(0, nb, sct, 0)

        plsc.subcore_barrier()

        @pl.when(sid < 10)
        def _():
            pltpu.sync_copy(acc_sh.at[pl.ds(sid * out_n, out_n)], zb_v)
            pltpu.sync_copy(zb_v, out_hbm.at[cid, pl.ds(sid * out_n, out_n)])

    return k(ec, dst2, zeros32)


def _sc_count(dst2, zeros8, ones8, E, N):
    per_w = E // _NW
    nout = per_w // _OCH
    nb = _OCH // _BATCH
    out_n = N // 10
    mesh = plsc.VectorSubcoreMesh(core_axis_name="c", subcore_axis_name="s")

    @functools.partial(
        pl.kernel,
        out_type=[jax.ShapeDtypeStruct((2, N, 8), F32)],
        mesh=mesh,
        scratch_types=[
            pltpu.VMEM((nb, _BATCH), jnp.int32),
            pltpu.VMEM((_BATCH, 8), F32),
            pltpu.VMEM((out_n, 8), F32),
            pltpu.VMEM_SHARED((N, 8), F32),
        ],
        compiler_params=pltpu.CompilerParams(use_tc_tiling_on_sc=False),
        interpret=False,
    )
    def k(di_hbm, z8_hbm, on8_hbm, cout_hbm, idx_v, ones_v, cb_v, cacc_sh):
        cid = lax.axis_index("c")
        sid = lax.axis_index("s")
        wid = sid * 2 + cid

        @pl.when(sid < 10)
        def _():
            pltpu.sync_copy(z8_hbm, cb_v)
            pltpu.sync_copy(cb_v, cacc_sh.at[pl.ds(sid * out_n, out_n)])
        pltpu.sync_copy(on8_hbm, ones_v)
        plsc.subcore_barrier()

        for o in range(nout):
            ch = wid * nout + o
            pltpu.sync_copy(di_hbm.at[ch], idx_v)

            def sct(j, _):
                pltpu.sync_copy(ones_v, cacc_sh.at[idx_v.at[j]], add=True)
                return 0

            lax.fori_loop(0, nb, sct, 0)

        plsc.subcore_barrier()

        @pl.when(sid < 10)
        def _():
            pltpu.sync_copy(cacc_sh.at[pl.ds(sid * out_n, out_n)], cb_v)
            pltpu.sync_copy(cb_v, cout_hbm.at[cid, pl.ds(sid * out_n, out_n)])

    return k(dst2, zeros8, ones8)


def _conv_body(has_dense, ep_ref, gi_ref, gj_ref, ud_ref,
               dw1, db1, dw2, db2,
               w1a, w1u, b1, w2, b2, w3, b3,
               ec_ref, eo_ref):
    ep = ep_ref[...]
    if has_dense:
        ed = _sp(_dot(_sp(_dot(ep, dw1[...]) + db1[...]), dw2[...]) + db2[...])
    else:
        ed = ep
    x = jnp.concatenate([gi_ref[...], gj_ref[...], ed], axis=1)
    h = _sp(_dot(x, w1a[...]) + _dot(ud_ref[...], w1u[...]) + b1[...])
    h = _sp(_dot(h, w2[...]) + b2[...])
    ec = _sp(_dot(h, w3[...]) + b3[...])
    ec_ref[...] = ec
    eo_ref[...] = ec + ep


def _conv_edge(e, gi, gj, ud, dense, conv, rows=8000):
    E = e.shape[0]
    grid = E // rows
    has_dense = len(dense) > 0
    if has_dense:
        dw1, db1, dw2, db2 = dense[0], dense[1].reshape(1, -1), dense[2], dense[3].reshape(1, -1)
    else:
        dw1 = jnp.zeros((32, 64), F32); db1 = jnp.zeros((1, 64), F32)
        dw2 = jnp.zeros((64, 32), F32); db2 = jnp.zeros((1, 32), F32)
    w1, b1, w2, b2, w3, b3 = conv
    w1a, w1u = w1[:96], w1[96:]
    b1 = b1.reshape(1, -1); b2 = b2.reshape(1, -1); b3 = b3.reshape(1, -1)

    full = lambda a: pl.BlockSpec(a.shape, lambda i: tuple(0 for _ in a.shape))
    chunk = lambda a: pl.BlockSpec((rows, a.shape[1]), lambda i: (i, 0))
    return pl.pallas_call(
        functools.partial(_conv_body, has_dense),
        grid=(grid,),
        in_specs=[chunk(e), chunk(gi), chunk(gj), full(ud),
                  full(dw1), full(db1), full(dw2), full(db2),
                  full(w1a), full(w1u), full(b1), full(w2), full(b2),
                  full(w3), full(b3)],
        out_specs=[pl.BlockSpec((rows, 32), lambda i: (i, 0)),
                   pl.BlockSpec((rows, 32), lambda i: (i, 0))],
        out_shape=[jax.ShapeDtypeStruct((E, 32), F32),
                   jax.ShapeDtypeStruct((E, 32), F32)],
        interpret=False,
    )(e, gi, gj, ud, dw1, db1, dw2, db2, w1a, w1u, b1, w2, b2, w3, b3)


def _node_body(has_next, inv_e, inv_n,
               vp_ref, vd_ref, ep_ref, cp_ref, up_ref, ud_ref,
               wna, wnb, wnc, bn1, wn2, bn2, wn3, bn3,
               waa, wab, wac, ba1, wa2, ba2, wa3, ba3,
               ndw1, ndb1, ndw2, ndb2, adw1, adb1, adw2, adb2,
               *out_refs):
    esum = ep_ref[0] + ep_ref[1]
    cnt = cp_ref[0, :, 0:1] + cp_ref[1, :, 0:1]
    ve = esum / jnp.maximum(cnt, 1.0)
    vd = vd_ref[...]
    ud = ud_ref[...]
    h = _sp(_dot(vd, wna[...]) + _dot(ve, wnb[...]) + _dot(ud, wnc[...]) + bn1[...])
    h = _sp(_dot(h, wn2[...]) + bn2[...])
    vc = _sp(_dot(h, wn3[...]) + bn3[...])
    v_out = vc + vp_ref[...]

    me = jnp.sum(esum, axis=0, keepdims=True) * inv_e
    mv = jnp.sum(vc, axis=0, keepdims=True) * inv_n
    ha = _sp(_dot(ud, waa[...]) + _dot(me, wab[...]) + _dot(mv, wac[...]) + ba1[...])
    ha = _sp(_dot(ha, wa2[...]) + ba2[...])
    uc = _sp(_dot(ha, wa3[...]) + ba3[...])
    u_out = uc + up_ref[...]

    out_refs[0][...] = v_out
    out_refs[1][...] = u_out
    if has_next:
        out_refs[2][...] = _sp(_dot(_sp(_dot(v_out, ndw1[...]) + ndb1[...]), ndw2[...]) + ndb2[...])
        out_refs[3][...] = _sp(_dot(_sp(_dot(u_out, adw1[...]) + adb1[...]), adw2[...]) + adb2[...])


def _node_attr(v_prev, vd, ep, cp, u_prev, ud, blk, nxt, E, N):
    wn1, bn1, wn2, bn2, wn3, bn3 = blk['conv_node']
    wa1, ba1, wa2, ba2, wa3, ba3 = blk['conv_attr']
    wna, wnb, wnc = wn1[:32], wn1[32:64], wn1[64:]
    waa, wab, wac = wa1[:32], wa1[32:64], wa1[64:]
    has_next = nxt is not None
    if has_next:
        ndw1, ndb1, ndw2, ndb2 = (nxt['dense_node'][0], nxt['dense_node'][1].reshape(1, -1),
                                  nxt['dense_node'][2], nxt['dense_node'][3].reshape(1, -1))
        adw1, adb1, adw2, adb2 = (nxt['dense_attr'][0], nxt['dense_attr'][1].reshape(1, -1),
                                  nxt['dense_attr'][2], nxt['dense_attr'][3].reshape(1, -1))
    else:
        ndw1 = jnp.zeros((32, 64), F32); ndb1 = jnp.zeros((1, 64), F32)
        ndw2 = jnp.zeros((64, 32), F32); ndb2 = jnp.zeros((1, 32), F32)
        adw1, adb1, adw2, adb2 = ndw1, ndb1, ndw2, ndb2

    args = (v_prev, vd, ep, cp, u_prev, ud,
            wna, wnb, wnc, bn1.reshape(1, -1), wn2, bn2.reshape(1, -1), wn3, bn3.reshape(1, -1),
            waa, wab, wac, ba1.reshape(1, -1), wa2, ba2.reshape(1, -1), wa3, ba3.reshape(1, -1),
            ndw1, ndb1, ndw2, ndb2, adw1, adb1, adw2, adb2)
    full = lambda a: pl.BlockSpec(a.shape, lambda: tuple(0 for _ in a.shape))
    out_shape = [jax.ShapeDtypeStruct((N, 32), F32), jax.ShapeDtypeStruct((1, 32), F32)]
    if has_next:
        out_shape += [jax.ShapeDtypeStruct((N, 32), F32), jax.ShapeDtypeStruct((1, 32), F32)]
    res = pl.pallas_call(
        functools.partial(_node_body, has_next, 1.0 / E, 1.0 / N),
        in_specs=[full(a) for a in args],
        out_specs=[pl.BlockSpec(s.shape, lambda: tuple(0 for _ in s.shape)) for s in out_shape],
        out_shape=out_shape,
        interpret=False,
    )(*args)
    if has_next:
        return res[0], res[1], res[2], res[3]
    return res[0], res[1], None, None


def _lstm_step(q, h, c, wih_t, whh_t, b):
    g = _dot(q, wih_t) + _dot(h, whh_t) + b
    i, f, gg, o = g[:, 0:32], g[:, 32:64], g[:, 64:96], g[:, 96:128]
    c2 = _sig(f) * c + _sig(i) * jnp.tanh(gg)
    h2 = _sig(o) * jnp.tanh(c2)
    return h2, c2


def _s2s_body(nch, e_ref, v_ref, u_ref,
              nwih, nwhh, nb_, ewih, ewhh, eb_,
              ow1n, ow1e, ow1u, ob1, ow2, ob2, ow3, ob3,
              o_ref, nv_ref, m_ref, s_ref, r_ref, h_ref, c_ref):
    step = pl.program_id(0)

    @pl.when(step == 0)
    def _():
        vv = v_ref[...]
        q = jnp.zeros((1, 64), F32)
        h = jnp.zeros((1, 32), F32)
        c = jnp.zeros((1, 32), F32)
        for _ in range(2):
            h, c = _lstm_step(q, h, c, nwih[...], nwhh[...], nb_[...])
            z = jnp.sum(vv * h, axis=1, keepdims=True)
            zm = jnp.max(z, axis=0, keepdims=True)
            a = jnp.exp(z - zm)
            sa = jnp.sum(a, axis=0, keepdims=True)
            r = jnp.sum(vv * (a / sa), axis=0, keepdims=True)
            q = jnp.concatenate([h, r], axis=1)
        nv_ref[...] = q
        h1, c1 = _lstm_step(jnp.zeros((1, 64), F32), jnp.zeros((1, 32), F32),
                            jnp.zeros((1, 32), F32), ewih[...], ewhh[...], eb_[...])
        h_ref[...] = h1
        c_ref[...] = c1
        m_ref[...] = jnp.full((1, 1), -1e30, F32)
        s_ref[...] = jnp.zeros((1, 1), F32)
        r_ref[...] = jnp.zeros((1, 32), F32)

    ee = e_ref[...]
    h = h_ref[...]
    z = jnp.sum(ee * h, axis=1, keepdims=True)
    zm = jnp.max(z, axis=0, keepdims=True)
    m_old = m_ref[...]
    m_new = jnp.maximum(m_old, zm)
    corr = jnp.exp(m_old - m_new)
    p = jnp.exp(z - m_new)
    m_ref[...] = m_new
    s_ref[...] = s_ref[...] * corr + jnp.sum(p, axis=0, keepdims=True)
    r_ref[...] = r_ref[...] * corr + jnp.sum(ee * p, axis=0, keepdims=True)

    @pl.when(step == nch - 1)
    def _():
        r1 = r_ref[...] / s_ref[...]
        q1 = jnp.concatenate([h_ref[...], r1], axis=1)
        h2, c2 = _lstm_step(q1, h_ref[...], c_ref[...], ewih[...], ewhh[...], eb_[...])
        h_ref[...] = h2
        c_ref[...] = c2
        m_ref[...] = jnp.full((1, 1), -1e30, F32)
        s_ref[...] = jnp.zeros((1, 1), F32)
        r_ref[...] = jnp.zeros((1, 32), F32)

    @pl.when(step == 2 * nch - 1)
    def _():
        evec = jnp.concatenate([h_ref[...], r_ref[...] / s_ref[...]], axis=1)
        nvec = nv_ref[...]
        hh = _sp(_dot(nvec, ow1n[...]) + _dot(evec, ow1e[...])
                 + _dot(u_ref[...], ow1u[...]) + ob1[...])
        hh = _sp(_dot(hh, ow2[...]) + ob2[...])
        o_ref[...] = _dot(hh, ow3[...]) + ob3[...]


def _final(e, v, u, node_s2s, edge_s2s, out_p, rows=16000):
    E = e.shape[0]
    nch = E // rows
    nwih = node_s2s[0].T
    nwhh = node_s2s[1].T
    nb_ = (node_s2s[2] + node_s2s[3]).reshape(1, -1)
    ewih = edge_s2s[0].T
    ewhh = edge_s2s[1].T
    eb_ = (edge_s2s[2] + edge_s2s[3]).reshape(1, -1)
    ow1, ob1, ow2, ob2, ow3, ob3 = out_p
    ow1n, ow1e, ow1u = ow1[0:64], ow1[64:128], ow1[128:160]
    args = (e, v, u, nwih, nwhh, nb_, ewih, ewhh, eb_,
            ow1n, ow1e, ow1u, ob1.reshape(1, -1), ow2, ob2.reshape(1, -1),
            ow3, ob3.reshape(1, -1))
    full = lambda a: pl.BlockSpec(a.shape, lambda i: tuple(0 for _ in a.shape))
    specs = [pl.BlockSpec((rows, 32), lambda i: (i % nch, 0))] + [full(a) for a in args[1:]]
    return pl.pallas_call(
        functools.partial(_s2s_body, nch),
        grid=(2 * nch,),
        in_specs=specs,
        out_specs=pl.BlockSpec((1, 1), lambda i: (0, 0)),
        out_shape=jax.ShapeDtypeStruct((1, 1), F32),
        scratch_shapes=[
            pltpu.VMEM((1, 64), F32),
            pltpu.VMEM((1, 1), F32),
            pltpu.VMEM((1, 1), F32),
            pltpu.VMEM((1, 32), F32),
            pltpu.VMEM((1, 32), F32),
            pltpu.VMEM((1, 32), F32),
        ],
        interpret=False,
    )(*args)


def kernel(edge_feat, node_feat, graph_attr, edge_index, params):
    p = params
    E = edge_feat.shape[0]
    N = node_feat.shape[0]
    src2 = edge_index[0].reshape(E // _OCH, _OCH // _BATCH, _BATCH)
    dst2 = edge_index[1].reshape(E // _OCH, _OCH // _BATCH, _BATCH)

    ee = p['enc_edge']
    e = _encode(edge_feat, ee[0], ee[1].reshape(1, -1), ee[2], ee[3].reshape(1, -1), 8000)
    en = p['enc_node']
    v = _encode(node_feat, en[0], en[1].reshape(1, -1), en[2], en[3].reshape(1, -1), N)
    ea = p['enc_attr']
    u = _encode(graph_attr, ea[0], ea[1].reshape(1, -1), ea[2], ea[3].reshape(1, -1), 1)

    zeros32 = jnp.zeros((N // 10, 32), F32)
    zeros8 = jnp.zeros((N // 10, 8), F32)
    ones8 = jnp.ones((_BATCH, 8), F32)

    (cp,) = _sc_count(dst2, zeros8, ones8, E, N)
    vd, ud = v, u
    nblk = len(p['blocks'])
    for b, blk in enumerate(p['blocks']):
        gi, gj = _sc_gather(vd, src2, dst2, E)
        ec, e_new = _conv_edge(e, gi, gj, ud, blk['dense_edge'], blk['conv_edge'])
        (ep,) = _sc_scatter(ec, dst2, zeros32, E, N)
        nxt = p['blocks'][b + 1] if b + 1 < nblk else None
        v_new, u_new, vd, ud = _node_attr(v, vd, ep, cp, u, ud, blk, nxt, E, N)
        e, v, u = e_new, v_new, u_new

    return _final(e, v, u, p['node_s2s'], p['edge_s2s'], p['out'])


# speedup vs baseline: 2.3365x; 1.1101x over previous
import functools

import jax
import jax.numpy as jnp
from jax import lax
from jax.experimental import pallas as pl
from jax.experimental.pallas import tpu as pltpu
from jax.experimental.pallas import tpu_sc as plsc

F32 = jnp.float32


def _sp(x):
    return jnp.maximum(x, 0.0) + jnp.log1p(jnp.exp(-jnp.abs(x)))


def _sig(x):
    return 1.0 / (1.0 + jnp.exp(-x))


def _dot(a, b):
    return jnp.dot(a, b, preferred_element_type=F32)


def _enc_body(x_ref, w1_ref, b1_ref, w2_ref, b2_ref, o_ref):
    h = _sp(_dot(x_ref[...], w1_ref[...]) + b1_ref[...])
    o_ref[...] = _sp(_dot(h, w2_ref[...]) + b2_ref[...])


def _encode(x, w1, b1, w2, b2, rows):
    n, d = x.shape
    grid = n // rows
    return pl.pallas_call(
        _enc_body,
        grid=(grid,),
        in_specs=[
            pl.BlockSpec((rows, d), lambda i: (i, 0)),
            pl.BlockSpec(w1.shape, lambda i: (0, 0)),
            pl.BlockSpec(b1.shape, lambda i: (0, 0)),
            pl.BlockSpec(w2.shape, lambda i: (0, 0)),
            pl.BlockSpec(b2.shape, lambda i: (0, 0)),
        ],
        out_specs=pl.BlockSpec((rows, w2.shape[1]), lambda i: (i, 0)),
        out_shape=jax.ShapeDtypeStruct((n, w2.shape[1]), F32),
        interpret=False,
    )(x, w1, b1, w2, b2)


_BATCH = 100
_OCH = 2000
_NW = 32


def _sc_gather(v, src2, dst2, E):
    per_w = E // _NW
    nout = per_w // _OCH
    nb = _OCH // _BATCH
    mesh = plsc.VectorSubcoreMesh(core_axis_name="c", subcore_axis_name="s")

    @functools.partial(
        pl.kernel,
        out_type=[
            jax.ShapeDtypeStruct((E, 32), F32),
            jax.ShapeDtypeStruct((E, 32), F32),
        ],
        mesh=mesh,
        scratch_types=[
            pltpu.VMEM((nb, _BATCH), jnp.int32),
            pltpu.VMEM((_OCH, 32), F32),
            pltpu.SemaphoreType.DMA,
        ],
        compiler_params=pltpu.CompilerParams(use_tc_tiling_on_sc=False),
        interpret=False,
    )
    def k(v_hbm, si_hbm, di_hbm, gi_hbm, gj_hbm, idx_v, rows_v, sem):
        cid = lax.axis_index("c")
        sid = lax.axis_index("s")
        wid = sid * 2 + cid

        def outer(o, _):
            ch = wid * nout + o
            e0 = wid * per_w + o * _OCH

            def burst(out_hbm):
                descs = [
                    pltpu.async_copy(
                        v_hbm.at[idx_v.at[j]],
                        rows_v.at[pl.ds(j * _BATCH, _BATCH)],
                        sem,
                    )
                    for j in range(nb)
                ]
                for d in descs:
                    d.wait()
                pltpu.sync_copy(rows_v, out_hbm.at[pl.ds(e0, _OCH)])

            pltpu.sync_copy(si_hbm.at[ch], idx_v)
            burst(gi_hbm)
            pltpu.sync_copy(di_hbm.at[ch], idx_v)
            burst(gj_hbm)
            return 0

        lax.fori_loop(0, nout, outer, 0)

    return k(v, src2, dst2)


def _sc_scatter(ec, dst2, zeros32, E, N):
    per_w = E // _NW
    nout = per_w // _OCH
    nb = _OCH // _BATCH
    out_n = N // 10
    mesh = plsc.VectorSubcoreMesh(core_axis_name="c", subcore_axis_name="s")

    @functools.partial(
        pl.kernel,
        out_type=[jax.ShapeDtypeStruct((2, N, 32), F32)],
        mesh=mesh,
        scratch_types=[
            pltpu.VMEM((nb, _BATCH), jnp.int32),
            pltpu.VMEM((_OCH, 32), F32),
            pltpu.VMEM((out_n, 32), F32),
            pltpu.VMEM_SHARED((N, 32), F32),
        ],
        compiler_params=pltpu.CompilerParams(use_tc_tiling_on_sc=False),
        interpret=False,
    )
    def k(ec_hbm, di_hbm, z32_hbm, out_hbm, idx_v, rows_v, zb_v, acc_sh):
        cid = lax.axis_index("c")
        sid = lax.axis_index("s")
        wid = sid * 2 + cid

        @pl.when(sid < 10)
        def _():
            pltpu.sync_copy(z32_hbm, zb_v)
            pltpu.sync_copy(zb_v, acc_sh.at[pl.ds(sid * out_n, out_n)])
        plsc.subcore_barrier()

        for o in range(nout):
            ch = wid * nout + o
            e0 = wid * per_w + o * _OCH
            pltpu.sync_copy(di_hbm.at[ch], idx_v)
            pltpu.sync_copy(ec_hbm.at[pl.ds(e0, _OCH)], rows_v)

            def sct(j, _):
                pltpu.sync_copy(
                    rows_v.at[pl.ds(j * _BATCH, _BATCH)],
                    acc_sh.at[idx_v.at[j]],
                    add=True,
                )
                return 0

            lax.fori_loop(0, nb, sct, 0)

        plsc.subcore_barrier()

        @pl.when(sid < 10)
        def _():
            pltpu.sync_copy(acc_sh.at[pl.ds(sid * out_n, out_n)], zb_v)
            pltpu.sync_copy(zb_v, out_hbm.at[cid, pl.ds(sid * out_n, out_n)])

    return k(ec, dst2, zeros32)


def _sc_count(dst2, zeros8, ones8, E, N):
    per_w = E // _NW
    nout = per_w // _OCH
    nb = _OCH // _BATCH
    out_n = N // 10
    mesh = plsc.VectorSubcoreMesh(core_axis_name="c", subcore_axis_name="s")

    @functools.partial(
        pl.kernel,
        out_type=[jax.ShapeDtypeStruct((2, N, 8), F32)],
        mesh=mesh,
        scratch_types=[
            pltpu.VMEM((nb, _BATCH), jnp.int32),
            pltpu.VMEM((_BATCH, 8), F32),
            pltpu.VMEM((out_n, 8), F32),
            pltpu.VMEM_SHARED((N, 8), F32),
        ],
        compiler_params=pltpu.CompilerParams(use_tc_tiling_on_sc=False),
        interpret=False,
    )
    def k(di_hbm, z8_hbm, on8_hbm, cout_hbm, idx_v, ones_v, cb_v, cacc_sh):
        cid = lax.axis_index("c")
        sid = lax.axis_index("s")
        wid = sid * 2 + cid

        @pl.when(sid < 10)
        def _():
            pltpu.sync_copy(z8_hbm, cb_v)
            pltpu.sync_copy(cb_v, cacc_sh.at[pl.ds(sid * out_n, out_n)])
        pltpu.sync_copy(on8_hbm, ones_v)
        plsc.subcore_barrier()

        for o in range(nout):
            ch = wid * nout + o
            pltpu.sync_copy(di_hbm.at[ch], idx_v)

            def sct(j, _):
                pltpu.sync_copy(ones_v, cacc_sh.at[idx_v.at[j]], add=True)
                return 0

            lax.fori_loop(0, nb, sct, 0)

        plsc.subcore_barrier()

        @pl.when(sid < 10)
        def _():
            pltpu.sync_copy(cacc_sh.at[pl.ds(sid * out_n, out_n)], cb_v)
            pltpu.sync_copy(cb_v, cout_hbm.at[cid, pl.ds(sid * out_n, out_n)])

    return k(dst2, zeros8, ones8)


def _conv_body(has_dense, ep_ref, gi_ref, gj_ref, ud_ref,
               dw1, db1, dw2, db2,
               w1a, w1u, b1, w2, b2, w3, b3,
               ec_ref, eo_ref):
    ep = ep_ref[...]
    if has_dense:
        ed = _sp(_dot(_sp(_dot(ep, dw1[...]) + db1[...]), dw2[...]) + db2[...])
    else:
        ed = ep
    x = jnp.concatenate([gi_ref[...], gj_ref[...], ed], axis=1)
    h = _sp(_dot(x, w1a[...]) + _dot(ud_ref[...], w1u[...]) + b1[...])
    h = _sp(_dot(h, w2[...]) + b2[...])
    ec = _sp(_dot(h, w3[...]) + b3[...])
    ec_ref[...] = ec
    eo_ref[...] = ec + ep


def _conv_edge(e, gi, gj, ud, dense, conv, rows=8000):
    E = e.shape[0]
    grid = E // rows
    has_dense = len(dense) > 0
    if has_dense:
        dw1, db1, dw2, db2 = dense[0], dense[1].reshape(1, -1), dense[2], dense[3].reshape(1, -1)
    else:
        dw1 = jnp.zeros((32, 64), F32); db1 = jnp.zeros((1, 64), F32)
        dw2 = jnp.zeros((64, 32), F32); db2 = jnp.zeros((1, 32), F32)
    w1, b1, w2, b2, w3, b3 = conv
    w1a, w1u = w1[:96], w1[96:]
    b1 = b1.reshape(1, -1); b2 = b2.reshape(1, -1); b3 = b3.reshape(1, -1)

    full = lambda a: pl.BlockSpec(a.shape, lambda i: tuple(0 for _ in a.shape))
    chunk = lambda a: pl.BlockSpec((rows, a.shape[1]), lambda i: (i, 0))
    return pl.pallas_call(
        functools.partial(_conv_body, has_dense),
        grid=(grid,),
        in_specs=[chunk(e), chunk(gi), chunk(gj), full(ud),
                  full(dw1), full(db1), full(dw2), full(db2),
                  full(w1a), full(w1u), full(b1), full(w2), full(b2),
                  full(w3), full(b3)],
        out_specs=[pl.BlockSpec((rows, 32), lambda i: (i, 0)),
                   pl.BlockSpec((rows, 32), lambda i: (i, 0))],
        out_shape=[jax.ShapeDtypeStruct((E, 32), F32),
                   jax.ShapeDtypeStruct((E, 32), F32)],
        interpret=False,
    )(e, gi, gj, ud, dw1, db1, dw2, db2, w1a, w1u, b1, w2, b2, w3, b3)


def _node_body(has_next, inv_e, inv_n,
               vp_ref, vd_ref, ep_ref, cp_ref, up_ref, ud_ref,
               wna, wnb, wnc, bn1, wn2, bn2, wn3, bn3,
               waa, wab, wac, ba1, wa2, ba2, wa3, ba3,
               ndw1, ndb1, ndw2, ndb2, adw1, adb1, adw2, adb2,
               *out_refs):
    esum = ep_ref[0] + ep_ref[1]
    cnt = cp_ref[0, :, 0:1] + cp_ref[1, :, 0:1]
    ve = esum / jnp.maximum(cnt, 1.0)
    vd = vd_ref[...]
    ud = ud_ref[...]
    h = _sp(_dot(vd, wna[...]) + _dot(ve, wnb[...]) + _dot(ud, wnc[...]) + bn1[...])
    h = _sp(_dot(h, wn2[...]) + bn2[...])
    vc = _sp(_dot(h, wn3[...]) + bn3[...])
    v_out = vc + vp_ref[...]

    me = jnp.sum(esum, axis=0, keepdims=True) * inv_e
    mv = jnp.sum(vc, axis=0, keepdims=True) * inv_n
    ha = _sp(_dot(ud, waa[...]) + _dot(me, wab[...]) + _dot(mv, wac[...]) + ba1[...])
    ha = _sp(_dot(ha, wa2[...]) + ba2[...])
    uc = _sp(_dot(ha, wa3[...]) + ba3[...])
    u_out = uc + up_ref[...]

    out_refs[0][...] = v_out
    out_refs[1][...] = u_out
    if has_next:
        out_refs[2][...] = _sp(_dot(_sp(_dot(v_out, ndw1[...]) + ndb1[...]), ndw2[...]) + ndb2[...])
        out_refs[3][...] = _sp(_dot(_sp(_dot(u_out, adw1[...]) + adb1[...]), adw2[...]) + adb2[...])


def _node_attr(v_prev, vd, ep, cp, u_prev, ud, blk, nxt, E, N):
    wn1, bn1, wn2, bn2, wn3, bn3 = blk['conv_node']
    wa1, ba1, wa2, ba2, wa3, ba3 = blk['conv_attr']
    wna, wnb, wnc = wn1[:32], wn1[32:64], wn1[64:]
    waa, wab, wac = wa1[:32], wa1[32:64], wa1[64:]
    has_next = nxt is not None
    if has_next:
        ndw1, ndb1, ndw2, ndb2 = (nxt['dense_node'][0], nxt['dense_node'][1].reshape(1, -1),
                                  nxt['dense_node'][2], nxt['dense_node'][3].reshape(1, -1))
        adw1, adb1, adw2, adb2 = (nxt['dense_attr'][0], nxt['dense_attr'][1].reshape(1, -1),
                                  nxt['dense_attr'][2], nxt['dense_attr'][3].reshape(1, -1))
    else:
        ndw1 = jnp.zeros((32, 64), F32); ndb1 = jnp.zeros((1, 64), F32)
        ndw2 = jnp.zeros((64, 32), F32); ndb2 = jnp.zeros((1, 32), F32)
        adw1, adb1, adw2, adb2 = ndw1, ndb1, ndw2, ndb2

    args = (v_prev, vd, ep, cp, u_prev, ud,
            wna, wnb, wnc, bn1.reshape(1, -1), wn2, bn2.reshape(1, -1), wn3, bn3.reshape(1, -1),
            waa, wab, wac, ba1.reshape(1, -1), wa2, ba2.reshape(1, -1), wa3, ba3.reshape(1, -1),
            ndw1, ndb1, ndw2, ndb2, adw1, adb1, adw2, adb2)
    full = lambda a: pl.BlockSpec(a.shape, lambda: tuple(0 for _ in a.shape))
    out_shape = [jax.ShapeDtypeStruct((N, 32), F32), jax.ShapeDtypeStruct((1, 32), F32)]
    if has_next:
        out_shape += [jax.ShapeDtypeStruct((N, 32), F32), jax.ShapeDtypeStruct((1, 32), F32)]
    res = pl.pallas_call(
        functools.partial(_node_body, has_next, 1.0 / E, 1.0 / N),
        in_specs=[full(a) for a in args],
        out_specs=[pl.BlockSpec(s.shape, lambda: tuple(0 for _ in s.shape)) for s in out_shape],
        out_shape=out_shape,
        interpret=False,
    )(*args)
    if has_next:
        return res[0], res[1], res[2], res[3]
    return res[0], res[1], None, None


def _lstm_step(q, h, c, wih_t, whh_t, b):
    g = _dot(q, wih_t) + _dot(h, whh_t) + b
    i, f, gg, o = g[:, 0:32], g[:, 32:64], g[:, 64:96], g[:, 96:128]
    c2 = _sig(f) * c + _sig(i) * jnp.tanh(gg)
    h2 = _sig(o) * jnp.tanh(c2)
    return h2, c2


def _s2s_body(nch, e_ref, v_ref, u_ref,
              nwih, nwhh, nb_, ewih, ewhh, eb_,
              ow1n, ow1e, ow1u, ob1, ow2, ob2, ow3, ob3,
              o_ref, nv_ref, m_ref, s_ref, r_ref, h_ref, c_ref):
    step = pl.program_id(0)

    @pl.when(step == 0)
    def _():
        vv = v_ref[...]
        q = jnp.zeros((1, 64), F32)
        h = jnp.zeros((1, 32), F32)
        c = jnp.zeros((1, 32), F32)
        for _ in range(2):
            h, c = _lstm_step(q, h, c, nwih[...], nwhh[...], nb_[...])
            z = jnp.sum(vv * h, axis=1, keepdims=True)
            zm = jnp.max(z, axis=0, keepdims=True)
            a = jnp.exp(z - zm)
            sa = jnp.sum(a, axis=0, keepdims=True)
            r = jnp.sum(vv * (a / sa), axis=0, keepdims=True)
            q = jnp.concatenate([h, r], axis=1)
        nv_ref[...] = q
        h1, c1 = _lstm_step(jnp.zeros((1, 64), F32), jnp.zeros((1, 32), F32),
                            jnp.zeros((1, 32), F32), ewih[...], ewhh[...], eb_[...])
        h_ref[...] = h1
        c_ref[...] = c1
        m_ref[...] = jnp.full((1, 1), -1e30, F32)
        s_ref[...] = jnp.zeros((1, 1), F32)
        r_ref[...] = jnp.zeros((1, 32), F32)

    ee = e_ref[...]
    h = h_ref[...]
    z = jnp.sum(ee * h, axis=1, keepdims=True)
    zm = jnp.max(z, axis=0, keepdims=True)
    m_old = m_ref[...]
    m_new = jnp.maximum(m_old, zm)
    corr = jnp.exp(m_old - m_new)
    p = jnp.exp(z - m_new)
    m_ref[...] = m_new
    s_ref[...] = s_ref[...] * corr + jnp.sum(p, axis=0, keepdims=True)
    r_ref[...] = r_ref[...] * corr + jnp.sum(ee * p, axis=0, keepdims=True)

    @pl.when(step == nch - 1)
    def _():
        r1 = r_ref[...] / s_ref[...]
        q1 = jnp.concatenate([h_ref[...], r1], axis=1)
        h2, c2 = _lstm_step(q1, h_ref[...], c_ref[...], ewih[...], ewhh[...], eb_[...])
        h_ref[...] = h2
        c_ref[...] = c2
        m_ref[...] = jnp.full((1, 1), -1e30, F32)
        s_ref[...] = jnp.zeros((1, 1), F32)
        r_ref[...] = jnp.zeros((1, 32), F32)

    @pl.when(step == 2 * nch - 1)
    def _():
        evec = jnp.concatenate([h_ref[...], r_ref[...] / s_ref[...]], axis=1)
        nvec = nv_ref[...]
        hh = _sp(_dot(nvec, ow1n[...]) + _dot(evec, ow1e[...])
                 + _dot(u_ref[...], ow1u[...]) + ob1[...])
        hh = _sp(_dot(hh, ow2[...]) + ob2[...])
        o_ref[...] = _dot(hh, ow3[...]) + ob3[...]


def _final(e, v, u, node_s2s, edge_s2s, out_p, rows=16000):
    E = e.shape[0]
    nch = E // rows
    nwih = node_s2s[0].T
    nwhh = node_s2s[1].T
    nb_ = (node_s2s[2] + node_s2s[3]).reshape(1, -1)
    ewih = edge_s2s[0].T
    ewhh = edge_s2s[1].T
    eb_ = (edge_s2s[2] + edge_s2s[3]).reshape(1, -1)
    ow1, ob1, ow2, ob2, ow3, ob3 = out_p
    ow1n, ow1e, ow1u = ow1[0:64], ow1[64:128], ow1[128:160]
    args = (e, v, u, nwih, nwhh, nb_, ewih, ewhh, eb_,
            ow1n, ow1e, ow1u, ob1.reshape(1, -1), ow2, ob2.reshape(1, -1),
            ow3, ob3.reshape(1, -1))
    full = lambda a: pl.BlockSpec(a.shape, lambda i: tuple(0 for _ in a.shape))
    specs = [pl.BlockSpec((rows, 32), lambda i: (i % nch, 0))] + [full(a) for a in args[1:]]
    return pl.pallas_call(
        functools.partial(_s2s_body, nch),
        grid=(2 * nch,),
        in_specs=specs,
        out_specs=pl.BlockSpec((1, 1), lambda i: (0, 0)),
        out_shape=jax.ShapeDtypeStruct((1, 1), F32),
        scratch_shapes=[
            pltpu.VMEM((1, 64), F32),
            pltpu.VMEM((1, 1), F32),
            pltpu.VMEM((1, 1), F32),
            pltpu.VMEM((1, 32), F32),
            pltpu.VMEM((1, 32), F32),
            pltpu.VMEM((1, 32), F32),
        ],
        interpret=False,
    )(*args)


def kernel(edge_feat, node_feat, graph_attr, edge_index, params):
    p = params
    E = edge_feat.shape[0]
    N = node_feat.shape[0]
    src2 = edge_index[0].reshape(E // _OCH, _OCH // _BATCH, _BATCH)
    dst2 = edge_index[1].reshape(E // _OCH, _OCH // _BATCH, _BATCH)
    assert _OCH % _BATCH == 0 and E % (_NW * _OCH) == 0

    ee = p['enc_edge']
    e = _encode(edge_feat, ee[0], ee[1].reshape(1, -1), ee[2], ee[3].reshape(1, -1), 8000)
    en = p['enc_node']
    v = _encode(node_feat, en[0], en[1].reshape(1, -1), en[2], en[3].reshape(1, -1), N)
    ea = p['enc_attr']
    u = _encode(graph_attr, ea[0], ea[1].reshape(1, -1), ea[2], ea[3].reshape(1, -1), 1)

    zeros32 = jnp.zeros((N // 10, 32), F32)
    zeros8 = jnp.zeros((N // 10, 8), F32)
    ones8 = jnp.ones((_BATCH, 8), F32)

    (cp,) = _sc_count(dst2, zeros8, ones8, E, N)
    vd, ud = v, u
    nblk = len(p['blocks'])
    for b, blk in enumerate(p['blocks']):
        gi, gj = _sc_gather(vd, src2, dst2, E)
        ec, e_new = _conv_edge(e, gi, gj, ud, blk['dense_edge'], blk['conv_edge'])
        (ep,) = _sc_scatter(ec, dst2, zeros32, E, N)
        nxt = p['blocks'][b + 1] if b + 1 < nblk else None
        v_new, u_new, vd, ud = _node_attr(v, vd, ep, cp, u, ud, blk, nxt, E, N)
        e, v, u = e_new, v_new, u_new

    return _final(e, v, u, p['node_s2s'], p['edge_s2s'], p['out'])


# speedup vs baseline: 2.5551x; 1.0936x over previous
import functools

import jax
import jax.numpy as jnp
from jax import lax
from jax.experimental import pallas as pl
from jax.experimental.pallas import tpu as pltpu
from jax.experimental.pallas import tpu_sc as plsc

F32 = jnp.float32


_LOG2E = 1.4426950408889634
_LN2 = 0.6931471805599453


def _sp(x):
    t = jnp.exp2(-jnp.abs(x) * _LOG2E)
    return jnp.maximum(x, 0.0) + jnp.log2(1.0 + t) * _LN2


def _sig(x):
    return 1.0 / (1.0 + jnp.exp(-x))


def _dot(a, b):
    return jnp.dot(a, b, preferred_element_type=F32)


def _enc_body(x_ref, w1_ref, b1_ref, w2_ref, b2_ref, o_ref):
    h = _sp(_dot(x_ref[...], w1_ref[...]) + b1_ref[...])
    o_ref[...] = _sp(_dot(h, w2_ref[...]) + b2_ref[...])


def _encode(x, w1, b1, w2, b2, rows):
    n, d = x.shape
    grid = n // rows
    return pl.pallas_call(
        _enc_body,
        grid=(grid,),
        in_specs=[
            pl.BlockSpec((rows, d), lambda i: (i, 0)),
            pl.BlockSpec(w1.shape, lambda i: (0, 0)),
            pl.BlockSpec(b1.shape, lambda i: (0, 0)),
            pl.BlockSpec(w2.shape, lambda i: (0, 0)),
            pl.BlockSpec(b2.shape, lambda i: (0, 0)),
        ],
        out_specs=pl.BlockSpec((rows, w2.shape[1]), lambda i: (i, 0)),
        out_shape=jax.ShapeDtypeStruct((n, w2.shape[1]), F32),
        interpret=False,
    )(x, w1, b1, w2, b2)


_BATCH = 100
_OCH = 2000
_NW = 32


def _sc_gather(v, src2, dst2, E):
    per_w = E // _NW
    nout = per_w // _OCH
    nb = _OCH // _BATCH
    mesh = plsc.VectorSubcoreMesh(core_axis_name="c", subcore_axis_name="s")

    @functools.partial(
        pl.kernel,
        out_type=[
            jax.ShapeDtypeStruct((E, 32), F32),
            jax.ShapeDtypeStruct((E, 32), F32),
        ],
        mesh=mesh,
        scratch_types=[
            pltpu.VMEM((nb, _BATCH), jnp.int32),
            pltpu.VMEM((_OCH, 32), F32),
            pltpu.SemaphoreType.DMA,
        ],
        compiler_params=pltpu.CompilerParams(use_tc_tiling_on_sc=False),
        interpret=False,
    )
    def k(v_hbm, si_hbm, di_hbm, gi_hbm, gj_hbm, idx_v, rows_v, sem):
        cid = lax.axis_index("c")
        sid = lax.axis_index("s")
        wid = sid * 2 + cid

        def outer(o, _):
            ch = wid * nout + o
            e0 = wid * per_w + o * _OCH

            def burst(out_hbm):
                descs = [
                    pltpu.async_copy(
                        v_hbm.at[idx_v.at[j]],
                        rows_v.at[pl.ds(j * _BATCH, _BATCH)],
                        sem,
                    )
                    for j in range(nb)
                ]
                for d in descs:
                    d.wait()
                pltpu.sync_copy(rows_v, out_hbm.at[pl.ds(e0, _OCH)])

            pltpu.sync_copy(si_hbm.at[ch], idx_v)
            burst(gi_hbm)
            pltpu.sync_copy(di_hbm.at[ch], idx_v)
            burst(gj_hbm)
            return 0

        lax.fori_loop(0, nout, outer, 0)

    return k(v, src2, dst2)


def _sc_scatter(ec, dst2, zeros32, E, N):
    per_w = E // _NW
    nout = per_w // _OCH
    nb = _OCH // _BATCH
    out_n = N // 10
    mesh = plsc.VectorSubcoreMesh(core_axis_name="c", subcore_axis_name="s")

    @functools.partial(
        pl.kernel,
        out_type=[jax.ShapeDtypeStruct((2, N, 32), F32)],
        mesh=mesh,
        scratch_types=[
            pltpu.VMEM((nb, _BATCH), jnp.int32),
            pltpu.VMEM((_OCH, 32), F32),
            pltpu.VMEM((out_n, 32), F32),
            pltpu.VMEM_SHARED((N, 32), F32),
        ],
        compiler_params=pltpu.CompilerParams(use_tc_tiling_on_sc=False),
        interpret=False,
    )
    def k(ec_hbm, di_hbm, z32_hbm, out_hbm, idx_v, rows_v, zb_v, acc_sh):
        cid = lax.axis_index("c")
        sid = lax.axis_index("s")
        wid = sid * 2 + cid

        @pl.when(sid < 10)
        def _():
            pltpu.sync_copy(z32_hbm, zb_v)
            pltpu.sync_copy(zb_v, acc_sh.at[pl.ds(sid * out_n, out_n)])
        plsc.subcore_barrier()

        for o in range(nout):
            ch = wid * nout + o
            e0 = wid * per_w + o * _OCH
            pltpu.sync_copy(di_hbm.at[ch], idx_v)
            pltpu.sync_copy(ec_hbm.at[pl.ds(e0, _OCH)], rows_v)

            def sct(j, _):
                pltpu.sync_copy(
                    rows_v.at[pl.ds(j * _BATCH, _BATCH)],
                    acc_sh.at[idx_v.at[j]],
                    add=True,
                )
                return 0

            lax.fori_loop(0, nb, sct, 0)

        plsc.subcore_barrier()

        @pl.when(sid < 10)
        def _():
            pltpu.sync_copy(acc_sh.at[pl.ds(sid * out_n, out_n)], zb_v)
            pltpu.sync_copy(zb_v, out_hbm.at[cid, pl.ds(sid * out_n, out_n)])

    return k(ec, dst2, zeros32)


def _sc_count(dst2, zeros8, ones8, E, N):
    per_w = E // _NW
    nout = per_w // _OCH
    nb = _OCH // _BATCH
    out_n = N // 10
    mesh = plsc.VectorSubcoreMesh(core_axis_name="c", subcore_axis_name="s")

    @functools.partial(
        pl.kernel,
        out_type=[jax.ShapeDtypeStruct((2, N, 8), F32)],
        mesh=mesh,
        scratch_types=[
            pltpu.VMEM((nb, _BATCH), jnp.int32),
            pltpu.VMEM((_BATCH, 8), F32),
            pltpu.VMEM((out_n, 8), F32),
            pltpu.VMEM_SHARED((N, 8), F32),
        ],
        compiler_params=pltpu.CompilerParams(use_tc_tiling_on_sc=False),
        interpret=False,
    )
    def k(di_hbm, z8_hbm, on8_hbm, cout_hbm, idx_v, ones_v, cb_v, cacc_sh):
        cid = lax.axis_index("c")
        sid = lax.axis_index("s")
        wid = sid * 2 + cid

        @pl.when(sid < 10)
        def _():
            pltpu.sync_copy(z8_hbm, cb_v)
            pltpu.sync_copy(cb_v, cacc_sh.at[pl.ds(sid * out_n, out_n)])
        pltpu.sync_copy(on8_hbm, ones_v)
        plsc.subcore_barrier()

        for o in range(nout):
            ch = wid * nout + o
            pltpu.sync_copy(di_hbm.at[ch], idx_v)

            def sct(j, _):
                pltpu.sync_copy(ones_v, cacc_sh.at[idx_v.at[j]], add=True)
                return 0

            lax.fori_loop(0, nb, sct, 0)

        plsc.subcore_barrier()

        @pl.when(sid < 10)
        def _():
            pltpu.sync_copy(cacc_sh.at[pl.ds(sid * out_n, out_n)], cb_v)
            pltpu.sync_copy(cb_v, cout_hbm.at[cid, pl.ds(sid * out_n, out_n)])

    return k(dst2, zeros8, ones8)


def _conv_body(has_dense, ep_ref, gi_ref, gj_ref, ud_ref,
               dw1, db1, dw2, db2,
               w1a, w1u, b1, w2, b2, w3, b3,
               ec_ref, eo_ref):
    ep = ep_ref[...]
    if has_dense:
        ed = _sp(_dot(_sp(_dot(ep, dw1[...]) + db1[...]), dw2[...]) + db2[...])
    else:
        ed = ep
    x = jnp.concatenate([gi_ref[...], gj_ref[...], ed], axis=1)
    h = _sp(_dot(x, w1a[...]) + _dot(ud_ref[...], w1u[...]) + b1[...])
    h = _sp(_dot(h, w2[...]) + b2[...])
    ec = _sp(_dot(h, w3[...]) + b3[...])
    ec_ref[...] = ec
    eo_ref[...] = ec + ep


def _conv_edge(e, gi, gj, ud, dense, conv, rows=8000):
    E = e.shape[0]
    grid = E // rows
    has_dense = len(dense) > 0
    if has_dense:
        dw1, db1, dw2, db2 = dense[0], dense[1].reshape(1, -1), dense[2], dense[3].reshape(1, -1)
    else:
        dw1 = jnp.zeros((32, 64), F32); db1 = jnp.zeros((1, 64), F32)
        dw2 = jnp.zeros((64, 32), F32); db2 = jnp.zeros((1, 32), F32)
    w1, b1, w2, b2, w3, b3 = conv
    w1a, w1u = w1[:96], w1[96:]
    b1 = b1.reshape(1, -1); b2 = b2.reshape(1, -1); b3 = b3.reshape(1, -1)

    full = lambda a: pl.BlockSpec(a.shape, lambda i: tuple(0 for _ in a.shape))
    chunk = lambda a: pl.BlockSpec((rows, a.shape[1]), lambda i: (i, 0))
    return pl.pallas_call(
        functools.partial(_conv_body, has_dense),
        grid=(grid,),
        in_specs=[chunk(e), chunk(gi), chunk(gj), full(ud),
                  full(dw1), full(db1), full(dw2), full(db2),
                  full(w1a), full(w1u), full(b1), full(w2), full(b2),
                  full(w3), full(b3)],
        out_specs=[pl.BlockSpec((rows, 32), lambda i: (i, 0)),
                   pl.BlockSpec((rows, 32), lambda i: (i, 0))],
        out_shape=[jax.ShapeDtypeStruct((E, 32), F32),
                   jax.ShapeDtypeStruct((E, 32), F32)],
        interpret=False,
    )(e, gi, gj, ud, dw1, db1, dw2, db2, w1a, w1u, b1, w2, b2, w3, b3)


def _node_body(has_next, inv_e, inv_n,
               vp_ref, vd_ref, ep_ref, cp_ref, up_ref, ud_ref,
               wna, wnb, wnc, bn1, wn2, bn2, wn3, bn3,
               waa, wab, wac, ba1, wa2, ba2, wa3, ba3,
               ndw1, ndb1, ndw2, ndb2, adw1, adb1, adw2, adb2,
               *out_refs):
    esum = ep_ref[0] + ep_ref[1]
    cnt = cp_ref[0, :, 0:1] + cp_ref[1, :, 0:1]
    ve = esum / jnp.maximum(cnt, 1.0)
    vd = vd_ref[...]
    ud = ud_ref[...]
    h = _sp(_dot(vd, wna[...]) + _dot(ve, wnb[...]) + _dot(ud, wnc[...]) + bn1[...])
    h = _sp(_dot(h, wn2[...]) + bn2[...])
    vc = _sp(_dot(h, wn3[...]) + bn3[...])
    v_out = vc + vp_ref[...]

    me = jnp.sum(esum, axis=0, keepdims=True) * inv_e
    mv = jnp.sum(vc, axis=0, keepdims=True) * inv_n
    ha = _sp(_dot(ud, waa[...]) + _dot(me, wab[...]) + _dot(mv, wac[...]) + ba1[...])
    ha = _sp(_dot(ha, wa2[...]) + ba2[...])
    uc = _sp(_dot(ha, wa3[...]) + ba3[...])
    u_out = uc + up_ref[...]

    out_refs[0][...] = v_out
    out_refs[1][...] = u_out
    if has_next:
        out_refs[2][...] = _sp(_dot(_sp(_dot(v_out, ndw1[...]) + ndb1[...]), ndw2[...]) + ndb2[...])
        out_refs[3][...] = _sp(_dot(_sp(_dot(u_out, adw1[...]) + adb1[...]), adw2[...]) + adb2[...])


def _node_attr(v_prev, vd, ep, cp, u_prev, ud, blk, nxt, E, N):
    wn1, bn1, wn2, bn2, wn3, bn3 = blk['conv_node']
    wa1, ba1, wa2, ba2, wa3, ba3 = blk['conv_attr']
    wna, wnb, wnc = wn1[:32], wn1[32:64], wn1[64:]
    waa, wab, wac = wa1[:32], wa1[32:64], wa1[64:]
    has_next = nxt is not None
    if has_next:
        ndw1, ndb1, ndw2, ndb2 = (nxt['dense_node'][0], nxt['dense_node'][1].reshape(1, -1),
                                  nxt['dense_node'][2], nxt['dense_node'][3].reshape(1, -1))
        adw1, adb1, adw2, adb2 = (nxt['dense_attr'][0], nxt['dense_attr'][1].reshape(1, -1),
                                  nxt['dense_attr'][2], nxt['dense_attr'][3].reshape(1, -1))
    else:
        ndw1 = jnp.zeros((32, 64), F32); ndb1 = jnp.zeros((1, 64), F32)
        ndw2 = jnp.zeros((64, 32), F32); ndb2 = jnp.zeros((1, 32), F32)
        adw1, adb1, adw2, adb2 = ndw1, ndb1, ndw2, ndb2

    args = (v_prev, vd, ep, cp, u_prev, ud,
            wna, wnb, wnc, bn1.reshape(1, -1), wn2, bn2.reshape(1, -1), wn3, bn3.reshape(1, -1),
            waa, wab, wac, ba1.reshape(1, -1), wa2, ba2.reshape(1, -1), wa3, ba3.reshape(1, -1),
            ndw1, ndb1, ndw2, ndb2, adw1, adb1, adw2, adb2)
    full = lambda a: pl.BlockSpec(a.shape, lambda: tuple(0 for _ in a.shape))
    out_shape = [jax.ShapeDtypeStruct((N, 32), F32), jax.ShapeDtypeStruct((1, 32), F32)]
    if has_next:
        out_shape += [jax.ShapeDtypeStruct((N, 32), F32), jax.ShapeDtypeStruct((1, 32), F32)]
    res = pl.pallas_call(
        functools.partial(_node_body, has_next, 1.0 / E, 1.0 / N),
        in_specs=[full(a) for a in args],
        out_specs=[pl.BlockSpec(s.shape, lambda: tuple(0 for _ in s.shape)) for s in out_shape],
        out_shape=out_shape,
        interpret=False,
    )(*args)
    if has_next:
        return res[0], res[1], res[2], res[3]
    return res[0], res[1], None, None


def _lstm_step(q, h, c, wih_t, whh_t, b):
    g = _dot(q, wih_t) + _dot(h, whh_t) + b
    i, f, gg, o = g[:, 0:32], g[:, 32:64], g[:, 64:96], g[:, 96:128]
    c2 = _sig(f) * c + _sig(i) * jnp.tanh(gg)
    h2 = _sig(o) * jnp.tanh(c2)
    return h2, c2


def _s2s_body(nch, e_ref, v_ref, u_ref,
              nwih, nwhh, nb_, ewih, ewhh, eb_,
              ow1n, ow1e, ow1u, ob1, ow2, ob2, ow3, ob3,
              o_ref, nv_ref, m_ref, s_ref, r_ref, h_ref, c_ref):
    step = pl.program_id(0)

    @pl.when(step == 0)
    def _():
        vv = v_ref[...]
        q = jnp.zeros((1, 64), F32)
        h = jnp.zeros((1, 32), F32)
        c = jnp.zeros((1, 32), F32)
        for _ in range(2):
            h, c = _lstm_step(q, h, c, nwih[...], nwhh[...], nb_[...])
            z = jnp.sum(vv * h, axis=1, keepdims=True)
            zm = jnp.max(z, axis=0, keepdims=True)
            a = jnp.exp(z - zm)
            sa = jnp.sum(a, axis=0, keepdims=True)
            r = jnp.sum(vv * (a / sa), axis=0, keepdims=True)
            q = jnp.concatenate([h, r], axis=1)
        nv_ref[...] = q
        h1, c1 = _lstm_step(jnp.zeros((1, 64), F32), jnp.zeros((1, 32), F32),
                            jnp.zeros((1, 32), F32), ewih[...], ewhh[...], eb_[...])
        h_ref[...] = h1
        c_ref[...] = c1
        m_ref[...] = jnp.full((1, 1), -1e30, F32)
        s_ref[...] = jnp.zeros((1, 1), F32)
        r_ref[...] = jnp.zeros((1, 32), F32)

    ee = e_ref[...]
    h = h_ref[...]
    z = jnp.sum(ee * h, axis=1, keepdims=True)
    zm = jnp.max(z, axis=0, keepdims=True)
    m_old = m_ref[...]
    m_new = jnp.maximum(m_old, zm)
    corr = jnp.exp(m_old - m_new)
    p = jnp.exp(z - m_new)
    m_ref[...] = m_new
    s_ref[...] = s_ref[...] * corr + jnp.sum(p, axis=0, keepdims=True)
    r_ref[...] = r_ref[...] * corr + jnp.sum(ee * p, axis=0, keepdims=True)

    @pl.when(step == nch - 1)
    def _():
        r1 = r_ref[...] / s_ref[...]
        q1 = jnp.concatenate([h_ref[...], r1], axis=1)
        h2, c2 = _lstm_step(q1, h_ref[...], c_ref[...], ewih[...], ewhh[...], eb_[...])
        h_ref[...] = h2
        c_ref[...] = c2
        m_ref[...] = jnp.full((1, 1), -1e30, F32)
        s_ref[...] = jnp.zeros((1, 1), F32)
        r_ref[...] = jnp.zeros((1, 32), F32)

    @pl.when(step == 2 * nch - 1)
    def _():
        evec = jnp.concatenate([h_ref[...], r_ref[...] / s_ref[...]], axis=1)
        nvec = nv_ref[...]
        hh = _sp(_dot(nvec, ow1n[...]) + _dot(evec, ow1e[...])
                 + _dot(u_ref[...], ow1u[...]) + ob1[...])
        hh = _sp(_dot(hh, ow2[...]) + ob2[...])
        o_ref[...] = _dot(hh, ow3[...]) + ob3[...]


def _final(e, v, u, node_s2s, edge_s2s, out_p, rows=20000):
    E = e.shape[0]
    nch = E // rows
    nwih = node_s2s[0].T
    nwhh = node_s2s[1].T
    nb_ = (node_s2s[2] + node_s2s[3]).reshape(1, -1)
    ewih = edge_s2s[0].T
    ewhh = edge_s2s[1].T
    eb_ = (edge_s2s[2] + edge_s2s[3]).reshape(1, -1)
    ow1, ob1, ow2, ob2, ow3, ob3 = out_p
    ow1n, ow1e, ow1u = ow1[0:64], ow1[64:128], ow1[128:160]
    args = (e, v, u, nwih, nwhh, nb_, ewih, ewhh, eb_,
            ow1n, ow1e, ow1u, ob1.reshape(1, -1), ow2, ob2.reshape(1, -1),
            ow3, ob3.reshape(1, -1))
    full = lambda a: pl.BlockSpec(a.shape, lambda i: tuple(0 for _ in a.shape))
    specs = [pl.BlockSpec((rows, 32), lambda i: (i % nch, 0))] + [full(a) for a in args[1:]]
    return pl.pallas_call(
        functools.partial(_s2s_body, nch),
        grid=(2 * nch,),
        in_specs=specs,
        out_specs=pl.BlockSpec((1, 1), lambda i: (0, 0)),
        out_shape=jax.ShapeDtypeStruct((1, 1), F32),
        scratch_shapes=[
            pltpu.VMEM((1, 64), F32),
            pltpu.VMEM((1, 1), F32),
            pltpu.VMEM((1, 1), F32),
            pltpu.VMEM((1, 32), F32),
            pltpu.VMEM((1, 32), F32),
            pltpu.VMEM((1, 32), F32),
        ],
        interpret=False,
    )(*args)


def kernel(edge_feat, node_feat, graph_attr, edge_index, params):
    p = params
    E = edge_feat.shape[0]
    N = node_feat.shape[0]
    src2 = edge_index[0].reshape(E // _OCH, _OCH // _BATCH, _BATCH)
    dst2 = edge_index[1].reshape(E // _OCH, _OCH // _BATCH, _BATCH)
    assert _OCH % _BATCH == 0 and E % (_NW * _OCH) == 0

    ee = p['enc_edge']
    e = _encode(edge_feat, ee[0], ee[1].reshape(1, -1), ee[2], ee[3].reshape(1, -1), 8000)
    en = p['enc_node']
    v = _encode(node_feat, en[0], en[1].reshape(1, -1), en[2], en[3].reshape(1, -1), N)
    ea = p['enc_attr']
    u = _encode(graph_attr, ea[0], ea[1].reshape(1, -1), ea[2], ea[3].reshape(1, -1), 1)

    zeros32 = jnp.zeros((N // 10, 32), F32)
    zeros8 = jnp.zeros((N // 10, 8), F32)
    ones8 = jnp.ones((_BATCH, 8), F32)

    (cp,) = _sc_count(dst2, zeros8, ones8, E, N)
    vd, ud = v, u
    nblk = len(p['blocks'])
    for b, blk in enumerate(p['blocks']):
        gi, gj = _sc_gather(vd, src2, dst2, E)
        ec, e_new = _conv_edge(e, gi, gj, ud, blk['dense_edge'], blk['conv_edge'])
        (ep,) = _sc_scatter(ec, dst2, zeros32, E, N)
        nxt = p['blocks'][b + 1] if b + 1 < nblk else None
        v_new, u_new, vd, ud = _node_attr(v, vd, ep, cp, u, ud, blk, nxt, E, N)
        e, v, u = e_new, v_new, u_new

    return _final(e, v, u, p['node_s2s'], p['edge_s2s'], p['out'])


# speedup vs baseline: 2.6774x; 1.0479x over previous
import functools

import jax
import jax.numpy as jnp
from jax import lax
from jax.experimental import pallas as pl
from jax.experimental.pallas import tpu as pltpu
from jax.experimental.pallas import tpu_sc as plsc

F32 = jnp.float32


_LOG2E = 1.4426950408889634
_LN2 = 0.6931471805599453


def _sp(x):
    t = jnp.exp(-jnp.abs(x))
    return jnp.maximum(x, 0.0) + jnp.log(1.0 + t)


def _sig(x):
    return 1.0 / (1.0 + jnp.exp(-x))


def _dot(a, b):
    return jnp.dot(a, b, preferred_element_type=F32)


def _enc_body(x_ref, w1_ref, b1_ref, w2_ref, b2_ref, o_ref):
    h = _sp(_dot(x_ref[...], w1_ref[...]) + b1_ref[...])
    o_ref[...] = _sp(_dot(h, w2_ref[...]) + b2_ref[...])


def _encode(x, w1, b1, w2, b2, rows):
    n, d = x.shape
    grid = n // rows
    return pl.pallas_call(
        _enc_body,
        grid=(grid,),
        in_specs=[
            pl.BlockSpec((rows, d), lambda i: (i, 0)),
            pl.BlockSpec(w1.shape, lambda i: (0, 0)),
            pl.BlockSpec(b1.shape, lambda i: (0, 0)),
            pl.BlockSpec(w2.shape, lambda i: (0, 0)),
            pl.BlockSpec(b2.shape, lambda i: (0, 0)),
        ],
        out_specs=pl.BlockSpec((rows, w2.shape[1]), lambda i: (i, 0)),
        out_shape=jax.ShapeDtypeStruct((n, w2.shape[1]), F32),
        interpret=False,
    )(x, w1, b1, w2, b2)


_BATCH = 100
_OCH = 2000
_NW = 32


def _sc_gather(v, src2, dst2, E):
    per_w = E // _NW
    nout = per_w // _OCH
    nb = _OCH // _BATCH
    mesh = plsc.VectorSubcoreMesh(core_axis_name="c", subcore_axis_name="s")

    @functools.partial(
        pl.kernel,
        out_type=[
            jax.ShapeDtypeStruct((E, 32), F32),
            jax.ShapeDtypeStruct((E, 32), F32),
        ],
        mesh=mesh,
        scratch_types=[
            pltpu.VMEM((nb, _BATCH), jnp.int32),
            pltpu.VMEM((_OCH, 32), F32),
            pltpu.SemaphoreType.DMA,
        ],
        compiler_params=pltpu.CompilerParams(use_tc_tiling_on_sc=False),
        interpret=False,
    )
    def k(v_hbm, si_hbm, di_hbm, gi_hbm, gj_hbm, idx_v, rows_v, sem):
        cid = lax.axis_index("c")
        sid = lax.axis_index("s")
        wid = sid * 2 + cid

        def outer(o, _):
            ch = wid * nout + o
            e0 = wid * per_w + o * _OCH

            def burst(out_hbm):
                descs = [
                    pltpu.async_copy(
                        v_hbm.at[idx_v.at[j]],
                        rows_v.at[pl.ds(j * _BATCH, _BATCH)],
                        sem,
                    )
                    for j in range(nb)
                ]
                for d in descs:
                    d.wait()
                pltpu.sync_copy(rows_v, out_hbm.at[pl.ds(e0, _OCH)])

            pltpu.sync_copy(si_hbm.at[ch], idx_v)
            burst(gi_hbm)
            pltpu.sync_copy(di_hbm.at[ch], idx_v)
            burst(gj_hbm)
            return 0

        lax.fori_loop(0, nout, outer, 0)

    return k(v, src2, dst2)


def _sc_scatter(ec, dst2, zeros32, E, N):
    per_w = E // _NW
    nout = per_w // _OCH
    nb = _OCH // _BATCH
    out_n = N // 10
    mesh = plsc.VectorSubcoreMesh(core_axis_name="c", subcore_axis_name="s")

    @functools.partial(
        pl.kernel,
        out_type=[jax.ShapeDtypeStruct((2, N, 32), F32)],
        mesh=mesh,
        scratch_types=[
            pltpu.VMEM((nb, _BATCH), jnp.int32),
            pltpu.VMEM((_OCH, 32), F32),
            pltpu.VMEM((out_n, 32), F32),
            pltpu.VMEM_SHARED((N, 32), F32),
        ],
        compiler_params=pltpu.CompilerParams(use_tc_tiling_on_sc=False),
        interpret=False,
    )
    def k(ec_hbm, di_hbm, z32_hbm, out_hbm, idx_v, rows_v, zb_v, acc_sh):
        cid = lax.axis_index("c")
        sid = lax.axis_index("s")
        wid = sid * 2 + cid

        @pl.when(sid < 10)
        def _():
            pltpu.sync_copy(z32_hbm, zb_v)
            pltpu.sync_copy(zb_v, acc_sh.at[pl.ds(sid * out_n, out_n)])
        plsc.subcore_barrier()

        for o in range(nout):
            ch = wid * nout + o
            e0 = wid * per_w + o * _OCH
            pltpu.sync_copy(di_hbm.at[ch], idx_v)
            pltpu.sync_copy(ec_hbm.at[pl.ds(e0, _OCH)], rows_v)

            def sct(j, _):
                pltpu.sync_copy(
                    rows_v.at[pl.ds(j * _BATCH, _BATCH)],
                    acc_sh.at[idx_v.at[j]],
                    add=True,
                )
                return 0

            lax.fori_loop(0, nb, sct, 0)

        plsc.subcore_barrier()

        @pl.when(sid < 10)
        def _():
            pltpu.sync_copy(acc_sh.at[pl.ds(sid * out_n, out_n)], zb_v)
            pltpu.sync_copy(zb_v, out_hbm.at[cid, pl.ds(sid * out_n, out_n)])

    return k(ec, dst2, zeros32)


def _sc_count(dst2, zeros8, ones8, E, N):
    per_w = E // _NW
    nout = per_w // _OCH
    nb = _OCH // _BATCH
    out_n = N // 10
    mesh = plsc.VectorSubcoreMesh(core_axis_name="c", subcore_axis_name="s")

    @functools.partial(
        pl.kernel,
        out_type=[jax.ShapeDtypeStruct((2, N, 8), F32)],
        mesh=mesh,
        scratch_types=[
            pltpu.VMEM((nb, _BATCH), jnp.int32),
            pltpu.VMEM((_BATCH, 8), F32),
            pltpu.VMEM((out_n, 8), F32),
            pltpu.VMEM_SHARED((N, 8), F32),
        ],
        compiler_params=pltpu.CompilerParams(use_tc_tiling_on_sc=False),
        interpret=False,
    )
    def k(di_hbm, z8_hbm, on8_hbm, cout_hbm, idx_v, ones_v, cb_v, cacc_sh):
        cid = lax.axis_index("c")
        sid = lax.axis_index("s")
        wid = sid * 2 + cid

        @pl.when(sid < 10)
        def _():
            pltpu.sync_copy(z8_hbm, cb_v)
            pltpu.sync_copy(cb_v, cacc_sh.at[pl.ds(sid * out_n, out_n)])
        pltpu.sync_copy(on8_hbm, ones_v)
        plsc.subcore_barrier()

        for o in range(nout):
            ch = wid * nout + o
            pltpu.sync_copy(di_hbm.at[ch], idx_v)

            def sct(j, _):
                pltpu.sync_copy(ones_v, cacc_sh.at[idx_v.at[j]], add=True)
                return 0

            lax.fori_loop(0, nb, sct, 0)

        plsc.subcore_barrier()

        @pl.when(sid < 10)
        def _():
            pltpu.sync_copy(cacc_sh.at[pl.ds(sid * out_n, out_n)], cb_v)
            pltpu.sync_copy(cb_v, cout_hbm.at[cid, pl.ds(sid * out_n, out_n)])

    return k(dst2, zeros8, ones8)


def _conv_body(has_dense, ep_ref, gi_ref, gj_ref, ud_ref,
               dw1, db1, dw2, db2,
               w1a, w1u, b1, w2, b2, w3, b3,
               ec_ref, eo_ref):
    ep = ep_ref[...]
    if has_dense:
        ed = _sp(_dot(_sp(_dot(ep, dw1[...]) + db1[...]), dw2[...]) + db2[...])
    else:
        ed = ep
    x = jnp.concatenate([gi_ref[...], gj_ref[...], ed], axis=1)
    h = _sp(_dot(x, w1a[...]) + _dot(ud_ref[...], w1u[...]) + b1[...])
    h = _sp(_dot(h, w2[...]) + b2[...])
    ec = _sp(_dot(h, w3[...]) + b3[...])
    ec_ref[...] = ec
    eo_ref[...] = ec + ep


def _conv_edge(e, gi, gj, ud, dense, conv, rows=8000):
    E = e.shape[0]
    grid = E // rows
    has_dense = len(dense) > 0
    if has_dense:
        dw1, db1, dw2, db2 = dense[0], dense[1].reshape(1, -1), dense[2], dense[3].reshape(1, -1)
    else:
        dw1 = jnp.zeros((32, 64), F32); db1 = jnp.zeros((1, 64), F32)
        dw2 = jnp.zeros((64, 32), F32); db2 = jnp.zeros((1, 32), F32)
    w1, b1, w2, b2, w3, b3 = conv
    w1a, w1u = w1[:96], w1[96:]
    b1 = b1.reshape(1, -1); b2 = b2.reshape(1, -1); b3 = b3.reshape(1, -1)

    full = lambda a: pl.BlockSpec(a.shape, lambda i: tuple(0 for _ in a.shape))
    chunk = lambda a: pl.BlockSpec((rows, a.shape[1]), lambda i: (i, 0))
    return pl.pallas_call(
        functools.partial(_conv_body, has_dense),
        grid=(grid,),
        in_specs=[chunk(e), chunk(gi), chunk(gj), full(ud),
                  full(dw1), full(db1), full(dw2), full(db2),
                  full(w1a), full(w1u), full(b1), full(w2), full(b2),
                  full(w3), full(b3)],
        out_specs=[pl.BlockSpec((rows, 32), lambda i: (i, 0)),
                   pl.BlockSpec((rows, 32), lambda i: (i, 0))],
        out_shape=[jax.ShapeDtypeStruct((E, 32), F32),
                   jax.ShapeDtypeStruct((E, 32), F32)],
        interpret=False,
    )(e, gi, gj, ud, dw1, db1, dw2, db2, w1a, w1u, b1, w2, b2, w3, b3)


def _node_body(has_next, inv_e, inv_n,
               vp_ref, vd_ref, ep_ref, cp_ref, up_ref, ud_ref,
               wna, wnb, wnc, bn1, wn2, bn2, wn3, bn3,
               waa, wab, wac, ba1, wa2, ba2, wa3, ba3,
               ndw1, ndb1, ndw2, ndb2, adw1, adb1, adw2, adb2,
               *out_refs):
    esum = ep_ref[0] + ep_ref[1]
    cnt = cp_ref[0, :, 0:1] + cp_ref[1, :, 0:1]
    ve = esum / jnp.maximum(cnt, 1.0)
    vd = vd_ref[...]
    ud = ud_ref[...]
    h = _sp(_dot(vd, wna[...]) + _dot(ve, wnb[...]) + _dot(ud, wnc[...]) + bn1[...])
    h = _sp(_dot(h, wn2[...]) + bn2[...])
    vc = _sp(_dot(h, wn3[...]) + bn3[...])
    v_out = vc + vp_ref[...]

    me = jnp.sum(esum, axis=0, keepdims=True) * inv_e
    mv = jnp.sum(vc, axis=0, keepdims=True) * inv_n
    ha = _sp(_dot(ud, waa[...]) + _dot(me, wab[...]) + _dot(mv, wac[...]) + ba1[...])
    ha = _sp(_dot(ha, wa2[...]) + ba2[...])
    uc = _sp(_dot(ha, wa3[...]) + ba3[...])
    u_out = uc + up_ref[...]

    out_refs[0][...] = v_out
    out_refs[1][...] = u_out
    if has_next:
        out_refs[2][...] = _sp(_dot(_sp(_dot(v_out, ndw1[...]) + ndb1[...]), ndw2[...]) + ndb2[...])
        out_refs[3][...] = _sp(_dot(_sp(_dot(u_out, adw1[...]) + adb1[...]), adw2[...]) + adb2[...])


def _node_attr(v_prev, vd, ep, cp, u_prev, ud, blk, nxt, E, N):
    wn1, bn1, wn2, bn2, wn3, bn3 = blk['conv_node']
    wa1, ba1, wa2, ba2, wa3, ba3 = blk['conv_attr']
    wna, wnb, wnc = wn1[:32], wn1[32:64], wn1[64:]
    waa, wab, wac = wa1[:32], wa1[32:64], wa1[64:]
    has_next = nxt is not None
    if has_next:
        ndw1, ndb1, ndw2, ndb2 = (nxt['dense_node'][0], nxt['dense_node'][1].reshape(1, -1),
                                  nxt['dense_node'][2], nxt['dense_node'][3].reshape(1, -1))
        adw1, adb1, adw2, adb2 = (nxt['dense_attr'][0], nxt['dense_attr'][1].reshape(1, -1),
                                  nxt['dense_attr'][2], nxt['dense_attr'][3].reshape(1, -1))
    else:
        ndw1 = jnp.zeros((32, 64), F32); ndb1 = jnp.zeros((1, 64), F32)
        ndw2 = jnp.zeros((64, 32), F32); ndb2 = jnp.zeros((1, 32), F32)
        adw1, adb1, adw2, adb2 = ndw1, ndb1, ndw2, ndb2

    args = (v_prev, vd, ep, cp, u_prev, ud,
            wna, wnb, wnc, bn1.reshape(1, -1), wn2, bn2.reshape(1, -1), wn3, bn3.reshape(1, -1),
            waa, wab, wac, ba1.reshape(1, -1), wa2, ba2.reshape(1, -1), wa3, ba3.reshape(1, -1),
            ndw1, ndb1, ndw2, ndb2, adw1, adb1, adw2, adb2)
    full = lambda a: pl.BlockSpec(a.shape, lambda: tuple(0 for _ in a.shape))
    out_shape = [jax.ShapeDtypeStruct((N, 32), F32), jax.ShapeDtypeStruct((1, 32), F32)]
    if has_next:
        out_shape += [jax.ShapeDtypeStruct((N, 32), F32), jax.ShapeDtypeStruct((1, 32), F32)]
    res = pl.pallas_call(
        functools.partial(_node_body, has_next, 1.0 / E, 1.0 / N),
        in_specs=[full(a) for a in args],
        out_specs=[pl.BlockSpec(s.shape, lambda: tuple(0 for _ in s.shape)) for s in out_shape],
        out_shape=out_shape,
        interpret=False,
    )(*args)
    if has_next:
        return res[0], res[1], res[2], res[3]
    return res[0], res[1], None, None


def _lstm_step(q, h, c, wih_t, whh_t, b):
    g = _dot(q, wih_t) + _dot(h, whh_t) + b
    i, f, gg, o = g[:, 0:32], g[:, 32:64], g[:, 64:96], g[:, 96:128]
    c2 = _sig(f) * c + _sig(i) * jnp.tanh(gg)
    h2 = _sig(o) * jnp.tanh(c2)
    return h2, c2


def _s2s_body(nch, e_ref, v_ref, u_ref,
              nwih, nwhh, nb_, ewih, ewhh, eb_,
              ow1n, ow1e, ow1u, ob1, ow2, ob2, ow3, ob3,
              o_ref, nv_ref, m_ref, s_ref, r_ref, h_ref, c_ref):
    step = pl.program_id(0)

    @pl.when(step == 0)
    def _():
        vv = v_ref[...]
        q = jnp.zeros((1, 64), F32)
        h = jnp.zeros((1, 32), F32)
        c = jnp.zeros((1, 32), F32)
        for _ in range(2):
            h, c = _lstm_step(q, h, c, nwih[...], nwhh[...], nb_[...])
            z = jnp.sum(vv * h, axis=1, keepdims=True)
            zm = jnp.max(z, axis=0, keepdims=True)
            a = jnp.exp(z - zm)
            sa = jnp.sum(a, axis=0, keepdims=True)
            r = jnp.sum(vv * (a / sa), axis=0, keepdims=True)
            q = jnp.concatenate([h, r], axis=1)
        nv_ref[...] = q
        h1, c1 = _lstm_step(jnp.zeros((1, 64), F32), jnp.zeros((1, 32), F32),
                            jnp.zeros((1, 32), F32), ewih[...], ewhh[...], eb_[...])
        h_ref[...] = h1
        c_ref[...] = c1
        m_ref[...] = jnp.full((1, 1), -1e30, F32)
        s_ref[...] = jnp.zeros((1, 1), F32)
        r_ref[...] = jnp.zeros((1, 32), F32)

    ee = e_ref[...]
    h = h_ref[...]
    z = jnp.sum(ee * h, axis=1, keepdims=True)
    zm = jnp.max(z, axis=0, keepdims=True)
    m_old = m_ref[...]
    m_new = jnp.maximum(m_old, zm)
    corr = jnp.exp(m_old - m_new)
    p = jnp.exp(z - m_new)
    m_ref[...] = m_new
    s_ref[...] = s_ref[...] * corr + jnp.sum(p, axis=0, keepdims=True)
    r_ref[...] = r_ref[...] * corr + jnp.sum(ee * p, axis=0, keepdims=True)

    @pl.when(step == nch - 1)
    def _():
        r1 = r_ref[...] / s_ref[...]
        q1 = jnp.concatenate([h_ref[...], r1], axis=1)
        h2, c2 = _lstm_step(q1, h_ref[...], c_ref[...], ewih[...], ewhh[...], eb_[...])
        h_ref[...] = h2
        c_ref[...] = c2
        m_ref[...] = jnp.full((1, 1), -1e30, F32)
        s_ref[...] = jnp.zeros((1, 1), F32)
        r_ref[...] = jnp.zeros((1, 32), F32)

    @pl.when(step == 2 * nch - 1)
    def _():
        evec = jnp.concatenate([h_ref[...], r_ref[...] / s_ref[...]], axis=1)
        nvec = nv_ref[...]
        hh = _sp(_dot(nvec, ow1n[...]) + _dot(evec, ow1e[...])
                 + _dot(u_ref[...], ow1u[...]) + ob1[...])
        hh = _sp(_dot(hh, ow2[...]) + ob2[...])
        o_ref[...] = _dot(hh, ow3[...]) + ob3[...]


def _final(e, v, u, node_s2s, edge_s2s, out_p, rows=20000):
    E = e.shape[0]
    nch = E // rows
    nwih = node_s2s[0].T
    nwhh = node_s2s[1].T
    nb_ = (node_s2s[2] + node_s2s[3]).reshape(1, -1)
    ewih = edge_s2s[0].T
    ewhh = edge_s2s[1].T
    eb_ = (edge_s2s[2] + edge_s2s[3]).reshape(1, -1)
    ow1, ob1, ow2, ob2, ow3, ob3 = out_p
    ow1n, ow1e, ow1u = ow1[0:64], ow1[64:128], ow1[128:160]
    args = (e, v, u, nwih, nwhh, nb_, ewih, ewhh, eb_,
            ow1n, ow1e, ow1u, ob1.reshape(1, -1), ow2, ob2.reshape(1, -1),
            ow3, ob3.reshape(1, -1))
    full = lambda a: pl.BlockSpec(a.shape, lambda i: tuple(0 for _ in a.shape))
    specs = [pl.BlockSpec((rows, 32), lambda i: (i % nch, 0))] + [full(a) for a in args[1:]]
    return pl.pallas_call(
        functools.partial(_s2s_body, nch),
        grid=(2 * nch,),
        in_specs=specs,
        out_specs=pl.BlockSpec((1, 1), lambda i: (0, 0)),
        out_shape=jax.ShapeDtypeStruct((1, 1), F32),
        scratch_shapes=[
            pltpu.VMEM((1, 64), F32),
            pltpu.VMEM((1, 1), F32),
            pltpu.VMEM((1, 1), F32),
            pltpu.VMEM((1, 32), F32),
            pltpu.VMEM((1, 32), F32),
            pltpu.VMEM((1, 32), F32),
        ],
        interpret=False,
    )(*args)


def kernel(edge_feat, node_feat, graph_attr, edge_index, params):
    p = params
    E = edge_feat.shape[0]
    N = node_feat.shape[0]
    src2 = edge_index[0].reshape(E // _OCH, _OCH // _BATCH, _BATCH)
    dst2 = edge_index[1].reshape(E // _OCH, _OCH // _BATCH, _BATCH)
    assert _OCH % _BATCH == 0 and E % (_NW * _OCH) == 0

    ee = p['enc_edge']
    e = _encode(edge_feat, ee[0], ee[1].reshape(1, -1), ee[2], ee[3].reshape(1, -1), 8000)
    en = p['enc_node']
    v = _encode(node_feat, en[0], en[1].reshape(1, -1), en[2], en[3].reshape(1, -1), N)
    ea = p['enc_attr']
    u = _encode(graph_attr, ea[0], ea[1].reshape(1, -1), ea[2], ea[3].reshape(1, -1), 1)

    zeros32 = jnp.zeros((N // 10, 32), F32)
    zeros8 = jnp.zeros((N // 10, 8), F32)
    ones8 = jnp.ones((_BATCH, 8), F32)

    (cp,) = _sc_count(dst2, zeros8, ones8, E, N)
    vd, ud = v, u
    nblk = len(p['blocks'])
    for b, blk in enumerate(p['blocks']):
        gi, gj = _sc_gather(vd, src2, dst2, E)
        ec, e_new = _conv_edge(e, gi, gj, ud, blk['dense_edge'], blk['conv_edge'])
        (ep,) = _sc_scatter(ec, dst2, zeros32, E, N)
        nxt = p['blocks'][b + 1] if b + 1 < nblk else None
        v_new, u_new, vd, ud = _node_attr(v, vd, ep, cp, u, ud, blk, nxt, E, N)
        e, v, u = e_new, v_new, u_new

    return _final(e, v, u, p['node_s2s'], p['edge_s2s'], p['out'])


# speedup vs baseline: 2.7325x; 1.0206x over previous
import functools

import jax
import jax.numpy as jnp
from jax import lax
from jax.experimental import pallas as pl
from jax.experimental.pallas import tpu as pltpu
from jax.experimental.pallas import tpu_sc as plsc

F32 = jnp.float32


_LOG2E = 1.4426950408889634
_LN2 = 0.6931471805599453


def _sp(x):
    t = jnp.exp(-jnp.abs(x))
    return jnp.maximum(x, 0.0) + jnp.log(1.0 + t)


def _sig(x):
    return 1.0 / (1.0 + jnp.exp(-x))


def _dot(a, b):
    return jnp.dot(a, b, preferred_element_type=F32)


def _enc_body(x_ref, w1_ref, b1_ref, w2_ref, b2_ref, o_ref):
    h = _sp(_dot(x_ref[...], w1_ref[...]) + b1_ref[...])
    o_ref[...] = _sp(_dot(h, w2_ref[...]) + b2_ref[...])


def _encode(x, w1, b1, w2, b2, rows):
    n, d = x.shape
    grid = n // rows
    return pl.pallas_call(
        _enc_body,
        grid=(grid,),
        in_specs=[
            pl.BlockSpec((rows, d), lambda i: (i, 0)),
            pl.BlockSpec(w1.shape, lambda i: (0, 0)),
            pl.BlockSpec(b1.shape, lambda i: (0, 0)),
            pl.BlockSpec(w2.shape, lambda i: (0, 0)),
            pl.BlockSpec(b2.shape, lambda i: (0, 0)),
        ],
        out_specs=pl.BlockSpec((rows, w2.shape[1]), lambda i: (i, 0)),
        out_shape=jax.ShapeDtypeStruct((n, w2.shape[1]), F32),
        interpret=False,
    )(x, w1, b1, w2, b2)


_BATCH = 100
_OCH = 2000
_NW = 32


def _sc_gather(v, src2, dst2, E):
    per_w = E // _NW
    nout = per_w // _OCH
    nb = _OCH // _BATCH
    mesh = plsc.VectorSubcoreMesh(core_axis_name="c", subcore_axis_name="s")

    @functools.partial(
        pl.kernel,
        out_type=[
            jax.ShapeDtypeStruct((E, 32), F32),
            jax.ShapeDtypeStruct((E, 32), F32),
        ],
        mesh=mesh,
        scratch_types=[
            pltpu.VMEM((nb, _BATCH), jnp.int32),
            pltpu.VMEM((_OCH, 32), F32),
            pltpu.SemaphoreType.DMA,
        ],
        compiler_params=pltpu.CompilerParams(use_tc_tiling_on_sc=False),
        interpret=False,
    )
    def k(v_hbm, si_hbm, di_hbm, gi_hbm, gj_hbm, idx_v, rows_v, sem):
        cid = lax.axis_index("c")
        sid = lax.axis_index("s")
        wid = sid * 2 + cid

        def outer(o, _):
            ch = wid * nout + o
            e0 = wid * per_w + o * _OCH

            def burst(out_hbm):
                descs = [
                    pltpu.async_copy(
                        v_hbm.at[idx_v.at[j]],
                        rows_v.at[pl.ds(j * _BATCH, _BATCH)],
                        sem,
                    )
                    for j in range(nb)
                ]
                for d in descs:
                    d.wait()
                pltpu.sync_copy(rows_v, out_hbm.at[pl.ds(e0, _OCH)])

            pltpu.sync_copy(si_hbm.at[ch], idx_v)
            burst(gi_hbm)
            pltpu.sync_copy(di_hbm.at[ch], idx_v)
            burst(gj_hbm)
            return 0

        lax.fori_loop(0, nout, outer, 0)

    return k(v, src2, dst2)


def _sc_scatter(ec, dst2, zeros32, E, N):
    per_w = E // _NW
    nout = per_w // _OCH
    nb = _OCH // _BATCH
    out_n = N // 10
    mesh = plsc.VectorSubcoreMesh(core_axis_name="c", subcore_axis_name="s")

    @functools.partial(
        pl.kernel,
        out_type=[jax.ShapeDtypeStruct((2, N, 32), F32)],
        mesh=mesh,
        scratch_types=[
            pltpu.VMEM((nb, _BATCH), jnp.int32),
            pltpu.VMEM((_OCH, 32), F32),
            pltpu.VMEM((out_n, 32), F32),
            pltpu.VMEM_SHARED((N, 32), F32),
        ],
        compiler_params=pltpu.CompilerParams(use_tc_tiling_on_sc=False),
        interpret=False,
    )
    def k(ec_hbm, di_hbm, z32_hbm, out_hbm, idx_v, rows_v, zb_v, acc_sh):
        cid = lax.axis_index("c")
        sid = lax.axis_index("s")
        wid = sid * 2 + cid

        @pl.when(sid < 10)
        def _():
            pltpu.sync_copy(z32_hbm, zb_v)
            pltpu.sync_copy(zb_v, acc_sh.at[pl.ds(sid * out_n, out_n)])
        plsc.subcore_barrier()

        for o in range(nout):
            ch = wid * nout + o
            e0 = wid * per_w + o * _OCH
            pltpu.sync_copy(di_hbm.at[ch], idx_v)
            pltpu.sync_copy(ec_hbm.at[pl.ds(e0, _OCH)], rows_v)

            def sct(j, _):
                pltpu.sync_copy(
                    rows_v.at[pl.ds(j * _BATCH, _BATCH)],
                    acc_sh.at[idx_v.at[j]],
                    add=True,
                )
                return 0

            lax.fori_loop(0, nb, sct, 0)

        plsc.subcore_barrier()

        @pl.when(sid < 10)
        def _():
            pltpu.sync_copy(acc_sh.at[pl.ds(sid * out_n, out_n)], zb_v)
            pltpu.sync_copy(zb_v, out_hbm.at[cid, pl.ds(sid * out_n, out_n)])

    return k(ec, dst2, zeros32)


def _sc_count(dst2, zeros8, ones8, E, N):
    per_w = E // _NW
    nout = per_w // _OCH
    nb = _OCH // _BATCH
    out_n = N // 10
    mesh = plsc.VectorSubcoreMesh(core_axis_name="c", subcore_axis_name="s")

    @functools.partial(
        pl.kernel,
        out_type=[jax.ShapeDtypeStruct((2, N, 8), F32)],
        mesh=mesh,
        scratch_types=[
            pltpu.VMEM((nb, _BATCH), jnp.int32),
            pltpu.VMEM((_BATCH, 8), F32),
            pltpu.VMEM((out_n, 8), F32),
            pltpu.VMEM_SHARED((N, 8), F32),
        ],
        compiler_params=pltpu.CompilerParams(use_tc_tiling_on_sc=False),
        interpret=False,
    )
    def k(di_hbm, z8_hbm, on8_hbm, cout_hbm, idx_v, ones_v, cb_v, cacc_sh):
        cid = lax.axis_index("c")
        sid = lax.axis_index("s")
        wid = sid * 2 + cid

        @pl.when(sid < 10)
        def _():
            pltpu.sync_copy(z8_hbm, cb_v)
            pltpu.sync_copy(cb_v, cacc_sh.at[pl.ds(sid * out_n, out_n)])
        pltpu.sync_copy(on8_hbm, ones_v)
        plsc.subcore_barrier()

        for o in range(nout):
            ch = wid * nout + o
            pltpu.sync_copy(di_hbm.at[ch], idx_v)

            def sct(j, _):
                pltpu.sync_copy(ones_v, cacc_sh.at[idx_v.at[j]], add=True)
                return 0

            lax.fori_loop(0, nb, sct, 0)

        plsc.subcore_barrier()

        @pl.when(sid < 10)
        def _():
            pltpu.sync_copy(cacc_sh.at[pl.ds(sid * out_n, out_n)], cb_v)
            pltpu.sync_copy(cb_v, cout_hbm.at[cid, pl.ds(sid * out_n, out_n)])

    return k(dst2, zeros8, ones8)


def _conv_body(res_dense, has_s2s, ep_ref, gi_ref, gj_ref, ud_ref,
               dw1, db1, dw2, db2,
               w1a, w1u, b1, w2, b2, w3, b3,
               *rest):
    if has_s2s:
        ewih, ewhh, eb_, ec_ref, eo_ref, m_ref, s_ref, r_ref = rest
    else:
        ec_ref, eo_ref = rest
    ep = ep_ref[...]
    ed = _sp(_dot(_sp(_dot(ep, dw1[...]) + db1[...]), dw2[...]) + db2[...])
    x = jnp.concatenate([gi_ref[...], gj_ref[...], ed], axis=1)
    h = _sp(_dot(x, w1a[...]) + _dot(ud_ref[...], w1u[...]) + b1[...])
    h = _sp(_dot(h, w2[...]) + b2[...])
    ec = _sp(_dot(h, w3[...]) + b3[...])
    ec_ref[...] = ec
    eo = ec + (ed if res_dense else ep)
    eo_ref[...] = eo
    if has_s2s:
        pid = pl.program_id(0)
        h1, _c1 = _lstm_step(jnp.zeros((1, 64), F32), jnp.zeros((1, 32), F32),
                             jnp.zeros((1, 32), F32), ewih[...], ewhh[...], eb_[...])

        @pl.when(pid == 0)
        def _():
            m_ref[...] = jnp.full((1, 1), -1e30, F32)
            s_ref[...] = jnp.zeros((1, 1), F32)
            r_ref[...] = jnp.zeros((1, 32), F32)

        z = jnp.sum(eo * h1, axis=1, keepdims=True)
        zm = jnp.max(z, axis=0, keepdims=True)
        m_old = m_ref[...]
        m_new = jnp.maximum(m_old, zm)
        corr = jnp.exp(m_old - m_new)
        p = jnp.exp(z - m_new)
        m_ref[...] = m_new
        s_ref[...] = s_ref[...] * corr + jnp.sum(p, axis=0, keepdims=True)
        r_ref[...] = r_ref[...] * corr + jnp.sum(eo * p, axis=0, keepdims=True)


def _conv_edge(e, gi, gj, ud, dense, conv, res_from_dense=False, s2s=None,
               rows=8000):
    E = gi.shape[0]
    grid = E // rows
    dw1, db1, dw2, db2 = dense[0], dense[1].reshape(1, -1), dense[2], dense[3].reshape(1, -1)
    w1, b1, w2, b2, w3, b3 = conv
    w1a, w1u = w1[:96], w1[96:]
    b1 = b1.reshape(1, -1); b2 = b2.reshape(1, -1); b3 = b3.reshape(1, -1)

    full = lambda a: pl.BlockSpec(a.shape, lambda i: tuple(0 for _ in a.shape))
    chunk = lambda a: pl.BlockSpec((rows, a.shape[1]), lambda i: (i, 0))
    args = (e, gi, gj, ud, dw1, db1, dw2, db2, w1a, w1u, b1, w2, b2, w3, b3)
    in_specs = [chunk(e), chunk(gi), chunk(gj)] + [full(a) for a in args[3:]]
    out_specs = [pl.BlockSpec((rows, 32), lambda i: (i, 0)),
                 pl.BlockSpec((rows, 32), lambda i: (i, 0))]
    out_shape = [jax.ShapeDtypeStruct((E, 32), F32),
                 jax.ShapeDtypeStruct((E, 32), F32)]
    if s2s is not None:
        ewih = s2s[0].T
        ewhh = s2s[1].T
        eb_ = (s2s[2] + s2s[3]).reshape(1, -1)
        args = args + (ewih, ewhh, eb_)
        in_specs += [full(ewih), full(ewhh), full(eb_)]
        cfull = lambda shp: pl.BlockSpec(shp, lambda i: tuple(0 for _ in shp))
        out_specs += [cfull((1, 1)), cfull((1, 1)), cfull((1, 32))]
        out_shape += [jax.ShapeDtypeStruct((1, 1), F32),
                      jax.ShapeDtypeStruct((1, 1), F32),
                      jax.ShapeDtypeStruct((1, 32), F32)]
    return pl.pallas_call(
        functools.partial(_conv_body, res_from_dense, s2s is not None),
        grid=(grid,),
        in_specs=in_specs,
        out_specs=out_specs,
        out_shape=out_shape,
        interpret=False,
    )(*args)


def _node_body(has_next, inv_e, inv_n,
               vp_ref, vd_ref, ep_ref, cp_ref, up_ref, ud_ref,
               wna, wnb, wnc, bn1, wn2, bn2, wn3, bn3,
               waa, wab, wac, ba1, wa2, ba2, wa3, ba3,
               ndw1, ndb1, ndw2, ndb2, adw1, adb1, adw2, adb2,
               *out_refs):
    esum = ep_ref[0] + ep_ref[1]
    cnt = cp_ref[0, :, 0:1] + cp_ref[1, :, 0:1]
    ve = esum / jnp.maximum(cnt, 1.0)
    vd = vd_ref[...]
    ud = ud_ref[...]
    h = _sp(_dot(vd, wna[...]) + _dot(ve, wnb[...]) + _dot(ud, wnc[...]) + bn1[...])
    h = _sp(_dot(h, wn2[...]) + bn2[...])
    vc = _sp(_dot(h, wn3[...]) + bn3[...])
    v_out = vc + vp_ref[...]

    me = jnp.sum(esum, axis=0, keepdims=True) * inv_e
    mv = jnp.sum(vc, axis=0, keepdims=True) * inv_n
    ha = _sp(_dot(ud, waa[...]) + _dot(me, wab[...]) + _dot(mv, wac[...]) + ba1[...])
    ha = _sp(_dot(ha, wa2[...]) + ba2[...])
    uc = _sp(_dot(ha, wa3[...]) + ba3[...])
    u_out = uc + up_ref[...]

    out_refs[0][...] = v_out
    out_refs[1][...] = u_out
    if has_next:
        out_refs[2][...] = _sp(_dot(_sp(_dot(v_out, ndw1[...]) + ndb1[...]), ndw2[...]) + ndb2[...])
        out_refs[3][...] = _sp(_dot(_sp(_dot(u_out, adw1[...]) + adb1[...]), adw2[...]) + adb2[...])


def _node_attr(v_prev, vd, ep, cp, u_prev, ud, blk, nxt, E, N):
    wn1, bn1, wn2, bn2, wn3, bn3 = blk['conv_node']
    wa1, ba1, wa2, ba2, wa3, ba3 = blk['conv_attr']
    wna, wnb, wnc = wn1[:32], wn1[32:64], wn1[64:]
    waa, wab, wac = wa1[:32], wa1[32:64], wa1[64:]
    has_next = nxt is not None
    if has_next:
        ndw1, ndb1, ndw2, ndb2 = (nxt['dense_node'][0], nxt['dense_node'][1].reshape(1, -1),
                                  nxt['dense_node'][2], nxt['dense_node'][3].reshape(1, -1))
        adw1, adb1, adw2, adb2 = (nxt['dense_attr'][0], nxt['dense_attr'][1].reshape(1, -1),
                                  nxt['dense_attr'][2], nxt['dense_attr'][3].reshape(1, -1))
    else:
        ndw1 = jnp.zeros((32, 64), F32); ndb1 = jnp.zeros((1, 64), F32)
        ndw2 = jnp.zeros((64, 32), F32); ndb2 = jnp.zeros((1, 32), F32)
        adw1, adb1, adw2, adb2 = ndw1, ndb1, ndw2, ndb2

    args = (v_prev, vd, ep, cp, u_prev, ud,
            wna, wnb, wnc, bn1.reshape(1, -1), wn2, bn2.reshape(1, -1), wn3, bn3.reshape(1, -1),
            waa, wab, wac, ba1.reshape(1, -1), wa2, ba2.reshape(1, -1), wa3, ba3.reshape(1, -1),
            ndw1, ndb1, ndw2, ndb2, adw1, adb1, adw2, adb2)
    full = lambda a: pl.BlockSpec(a.shape, lambda: tuple(0 for _ in a.shape))
    out_shape = [jax.ShapeDtypeStruct((N, 32), F32), jax.ShapeDtypeStruct((1, 32), F32)]
    if has_next:
        out_shape += [jax.ShapeDtypeStruct((N, 32), F32), jax.ShapeDtypeStruct((1, 32), F32)]
    res = pl.pallas_call(
        functools.partial(_node_body, has_next, 1.0 / E, 1.0 / N),
        in_specs=[full(a) for a in args],
        out_specs=[pl.BlockSpec(s.shape, lambda: tuple(0 for _ in s.shape)) for s in out_shape],
        out_shape=out_shape,
        interpret=False,
    )(*args)
    if has_next:
        return res[0], res[1], res[2], res[3]
    return res[0], res[1], None, None


def _lstm_step(q, h, c, wih_t, whh_t, b):
    g = _dot(q, wih_t) + _dot(h, whh_t) + b
    i, f, gg, o = g[:, 0:32], g[:, 32:64], g[:, 64:96], g[:, 96:128]
    c2 = _sig(f) * c + _sig(i) * jnp.tanh(gg)
    h2 = _sig(o) * jnp.tanh(c2)
    return h2, c2


def _s2s_body(nch, e_ref, v_ref, u_ref, s1_ref, r1_ref,
              nwih, nwhh, nb_, ewih, ewhh, eb_,
              ow1n, ow1e, ow1u, ob1, ow2, ob2, ow3, ob3,
              o_ref, nv_ref, m_ref, s_ref, r_ref, h_ref, c_ref):
    step = pl.program_id(0)

    @pl.when(step == 0)
    def _():
        vv = v_ref[...]
        q = jnp.zeros((1, 64), F32)
        h = jnp.zeros((1, 32), F32)
        c = jnp.zeros((1, 32), F32)
        for _ in range(2):
            h, c = _lstm_step(q, h, c, nwih[...], nwhh[...], nb_[...])
            z = jnp.sum(vv * h, axis=1, keepdims=True)
            zm = jnp.max(z, axis=0, keepdims=True)
            a = jnp.exp(z - zm)
            sa = jnp.sum(a, axis=0, keepdims=True)
            r = jnp.sum(vv * (a / sa), axis=0, keepdims=True)
            q = jnp.concatenate([h, r], axis=1)
        nv_ref[...] = q
        h1, c1 = _lstm_step(jnp.zeros((1, 64), F32), jnp.zeros((1, 32), F32),
                            jnp.zeros((1, 32), F32), ewih[...], ewhh[...], eb_[...])
        r1 = r1_ref[...] / s1_ref[...]
        q1 = jnp.concatenate([h1, r1], axis=1)
        h2, c2 = _lstm_step(q1, h1, c1, ewih[...], ewhh[...], eb_[...])
        h_ref[...] = h2
        c_ref[...] = c2
        m_ref[...] = jnp.full((1, 1), -1e30, F32)
        s_ref[...] = jnp.zeros((1, 1), F32)
        r_ref[...] = jnp.zeros((1, 32), F32)

    ee = e_ref[...]
    h = h_ref[...]
    z = jnp.sum(ee * h, axis=1, keepdims=True)
    zm = jnp.max(z, axis=0, keepdims=True)
    m_old = m_ref[...]
    m_new = jnp.maximum(m_old, zm)
    corr = jnp.exp(m_old - m_new)
    p = jnp.exp(z - m_new)
    m_ref[...] = m_new
    s_ref[...] = s_ref[...] * corr + jnp.sum(p, axis=0, keepdims=True)
    r_ref[...] = r_ref[...] * corr + jnp.sum(ee * p, axis=0, keepdims=True)

    @pl.when(step == nch - 1)
    def _():
        evec = jnp.concatenate([h_ref[...], r_ref[...] / s_ref[...]], axis=1)
        nvec = nv_ref[...]
        hh = _sp(_dot(nvec, ow1n[...]) + _dot(evec, ow1e[...])
                 + _dot(u_ref[...], ow1u[...]) + ob1[...])
        hh = _sp(_dot(hh, ow2[...]) + ob2[...])
        o_ref[...] = _dot(hh, ow3[...]) + ob3[...]


def _final(e, v, u, s1, r1, node_s2s, edge_s2s, out_p, rows=20000):
    E = e.shape[0]
    nch = E // rows
    nwih = node_s2s[0].T
    nwhh = node_s2s[1].T
    nb_ = (node_s2s[2] + node_s2s[3]).reshape(1, -1)
    ewih = edge_s2s[0].T
    ewhh = edge_s2s[1].T
    eb_ = (edge_s2s[2] + edge_s2s[3]).reshape(1, -1)
    ow1, ob1, ow2, ob2, ow3, ob3 = out_p
    ow1n, ow1e, ow1u = ow1[0:64], ow1[64:128], ow1[128:160]
    args = (e, v, u, s1, r1, nwih, nwhh, nb_, ewih, ewhh, eb_,
            ow1n, ow1e, ow1u, ob1.reshape(1, -1), ow2, ob2.reshape(1, -1),
            ow3, ob3.reshape(1, -1))
    full = lambda a: pl.BlockSpec(a.shape, lambda i: tuple(0 for _ in a.shape))
    specs = [pl.BlockSpec((rows, 32), lambda i: (i, 0))] + [full(a) for a in args[1:]]
    return pl.pallas_call(
        functools.partial(_s2s_body, nch),
        grid=(nch,),
        in_specs=specs,
        out_specs=pl.BlockSpec((1, 1), lambda i: (0, 0)),
        out_shape=jax.ShapeDtypeStruct((1, 1), F32),
        scratch_shapes=[
            pltpu.VMEM((1, 64), F32),
            pltpu.VMEM((1, 1), F32),
            pltpu.VMEM((1, 1), F32),
            pltpu.VMEM((1, 32), F32),
            pltpu.VMEM((1, 32), F32),
            pltpu.VMEM((1, 32), F32),
        ],
        interpret=False,
    )(*args)


def kernel(edge_feat, node_feat, graph_attr, edge_index, params):
    p = params
    E = edge_feat.shape[0]
    N = node_feat.shape[0]
    src2 = edge_index[0].reshape(E // _OCH, _OCH // _BATCH, _BATCH)
    dst2 = edge_index[1].reshape(E // _OCH, _OCH // _BATCH, _BATCH)
    assert _OCH % _BATCH == 0 and E % (_NW * _OCH) == 0

    en = p['enc_node']
    v = _encode(node_feat, en[0], en[1].reshape(1, -1), en[2], en[3].reshape(1, -1), N)
    ea = p['enc_attr']
    u = _encode(graph_attr, ea[0], ea[1].reshape(1, -1), ea[2], ea[3].reshape(1, -1), 1)

    zeros32 = jnp.zeros((N // 10, 32), F32)
    zeros8 = jnp.zeros((N // 10, 8), F32)
    ones8 = jnp.ones((_BATCH, 8), F32)

    (cp,) = _sc_count(dst2, zeros8, ones8, E, N)
    e = edge_feat
    vd, ud = v, u
    nblk = len(p['blocks'])
    s1 = r1 = None
    for b, blk in enumerate(p['blocks']):
        gi, gj = _sc_gather(vd, src2, dst2, E)
        dense = p['enc_edge'] if b == 0 else blk['dense_edge']
        last = b + 1 == nblk
        res = _conv_edge(e, gi, gj, ud, dense, blk['conv_edge'],
                         res_from_dense=(b == 0),
                         s2s=p['edge_s2s'] if last else None)
        if last:
            ec, e_new, _m1, s1, r1 = res
        else:
            ec, e_new = res
        (ep,) = _sc_scatter(ec, dst2, zeros32, E, N)
        nxt = p['blocks'][b + 1] if b + 1 < nblk else None
        v_new, u_new, vd, ud = _node_attr(v, vd, ep, cp, u, ud, blk, nxt, E, N)
        e, v, u = e_new, v_new, u_new

    return _final(e, v, u, s1, r1, p['node_s2s'], p['edge_s2s'], p['out'])
